# Initial kernel scaffold; baseline (speedup 1.0000x reference)
import jax
import jax.numpy as jnp
from jax.experimental import pallas as pl


def kernel(atomic_number, edge_index, lg_edge_index, h, dnr, key_embedding, value_table, a, b, c, d):
    raise NotImplementedError("write your pallas kernel here")


# speedup vs baseline: 123.6631x; 123.6631x over previous
import numpy as np
import jax
import jax.numpy as jnp
from jax import lax
from jax.experimental import pallas as pl
from jax.experimental.pallas import tpu as pltpu
from jax.experimental.pallas import tpu_sc as plsc

_EPS = 1e-3
_NC, _NS = 2, 16


def _round_up(x, m):
    return (x + m - 1) // m * m


def _prep_tc(h2, dnr2, kh, scal):
    tch = h2.shape[0]
    na, hid = kh.shape[1], kh.shape[2]

    def body(h_ref, d_ref, k_ref, s_ref, sp_ref, w_ref):
        x = jnp.clip(h_ref[...], -_EPS, _EPS)
        theta = jnp.float32(np.pi / 2) - x - (x * x * x) * jnp.float32(1.0 / 6.0)
        d2 = d_ref[...] * d_ref[...]
        for hd in range(2):
            av = s_ref[0, hd]
            bv = s_ref[1, hd]
            cv = s_ref[2, hd]
            dv = s_ref[3, hd]
            ang = ((jnp.cos(av * theta + bv) + 1.0) * 0.5) ** cv
            rad = jnp.exp(-dv * d2)
            sp_ref[hd] = ang * rad
            k = k_ref[hd]
            w_ref[hd] = lax.dot_general(
                k, k, (((1,), (1,)), ((), ())),
                preferred_element_type=jnp.float32)

    return pl.pallas_call(
        body,
        in_specs=[
            pl.BlockSpec(memory_space=pltpu.VMEM),
            pl.BlockSpec(memory_space=pltpu.VMEM),
            pl.BlockSpec(memory_space=pltpu.VMEM),
            pl.BlockSpec(memory_space=pltpu.SMEM),
        ],
        out_shape=(
            jax.ShapeDtypeStruct((2, tch, 128), jnp.float32),
            jax.ShapeDtypeStruct((2, na, na), jnp.float32),
        ),
    )(h2, dnr2, kh, scal)


def _phase1_sc(an, src_p, dst_p, lsrc_p, ldst_p, sp0, sp1, wflat,
               n, na, e, ep, tp):
    ept = ep // _NS
    tt = tp // (_NC * _NS)
    nch = tt // 1024
    ot = e // _NS
    mesh = plsc.VectorSubcoreMesh(core_axis_name="c", subcore_axis_name="s")

    def body(an_h, src_h, dst_h, lsrc_h, ldst_h, sp0_h, sp1_h, w_h,
             sv_h, cnt_h,
             an_v, w_v, srcbuf, idsbuf, lsrc2d, ldst2d, sp0buf, sp1buf,
             kibuf, kjbuf, v0b2, v1b2, ones_v, zbuf,
             ks_sh, kd_sh, s0_sh, s1_sh, c_sh):
        c = lax.axis_index("c")
        s = lax.axis_index("s")
        pltpu.sync_copy(an_h, an_v)
        pltpu.sync_copy(w_h, w_v)

        def fill_ids(eh, sh):
            def blk(bi, _):
                ebase = s * ept + bi * 2048
                pltpu.sync_copy(eh.at[pl.ds(ebase, 2048)], srcbuf)

                def gg(g, _):
                    idx = srcbuf[pl.ds(g * 16, 16)]
                    idsbuf[pl.ds(g * 16, 16)] = plsc.load_gather(an_v, [idx])
                    return _

                lax.fori_loop(0, 128, gg, None)
                pltpu.sync_copy(idsbuf, sh.at[pl.ds(ebase, 2048)])
                return _

            lax.fori_loop(0, ept // 2048, blk, None)

        fill_ids(src_h, ks_sh)
        fill_ids(dst_h, kd_sh)

        def zz(k, _):
            zbuf[pl.ds(k * 16, 16)] = jnp.zeros((16,), jnp.float32)
            return _

        lax.fori_loop(0, 128, zz, None)

        def oo(k, _):
            ones_v[pl.ds(k * 16, 16)] = jnp.ones((16,), jnp.float32)
            return _

        lax.fori_loop(0, 8, oo, None)

        def z2(k, _):
            off = s * (ep // _NS) + k * 2048
            pltpu.sync_copy(zbuf, s0_sh.at[pl.ds(off, 2048)])
            pltpu.sync_copy(zbuf, s1_sh.at[pl.ds(off, 2048)])
            pltpu.sync_copy(zbuf, c_sh.at[pl.ds(off, 2048)])
            return _

        lax.fori_loop(0, ep // _NS // 2048, z2, None)
        plsc.subcore_barrier()

        tstart = c * (tp // 2) + s * tt

        def chunk(ch, _):
            tb = tstart + ch * 1024

            def ldrow(r, _):
                pltpu.sync_copy(lsrc_h.at[pl.ds(tb + r * 128, 128)],
                                lsrc2d.at[r])
                pltpu.sync_copy(ldst_h.at[pl.ds(tb + r * 128, 128)],
                                ldst2d.at[r])
                return _

            lax.fori_loop(0, 8, ldrow, None)
            pltpu.sync_copy(sp0_h.at[pl.ds(tb, 1024)], sp0buf)
            pltpu.sync_copy(sp1_h.at[pl.ds(tb, 1024)], sp1buf)

            def sub(si, _):
                pltpu.sync_copy(ks_sh.at[lsrc2d.at[si]], kibuf)
                pltpu.sync_copy(kd_sh.at[ldst2d.at[si]], kjbuf)

                def gg(g, _):
                    i = kibuf[pl.ds(g * 16, 16)]
                    j = kjbuf[pl.ds(g * 16, 16)]
                    fidx = i * na + j
                    w0 = plsc.load_gather(w_v, [fidx])
                    w1 = plsc.load_gather(w_v, [fidx + na * na])
                    o = si * 128 + g * 16
                    v0b2[si, pl.ds(g * 16, 16)] = w0 * sp0buf[pl.ds(o, 16)]
                    v1b2[si, pl.ds(g * 16, 16)] = w1 * sp1buf[pl.ds(o, 16)]
                    return _

                lax.fori_loop(0, 8, gg, None)
                pltpu.sync_copy(v0b2.at[si], s0_sh.at[lsrc2d.at[si]],
                                add=True)
                pltpu.sync_copy(v1b2.at[si], s1_sh.at[lsrc2d.at[si]],
                                add=True)
                pltpu.sync_copy(ones_v, c_sh.at[lsrc2d.at[si]], add=True)
                return _

            lax.fori_loop(0, 8, sub, None)
            return _

        lax.fori_loop(0, nch, chunk, None)
        plsc.subcore_barrier()

        ostart = s * 10240
        olen = jnp.minimum(10240, e - ostart)

        def out(k, _):
            off = ostart + k * 1280
            pltpu.sync_copy(s0_sh.at[pl.ds(off, 1280)],
                            sv_h.at[c].at[0].at[pl.ds(off, 1280)])
            pltpu.sync_copy(s1_sh.at[pl.ds(off, 1280)],
                            sv_h.at[c].at[1].at[pl.ds(off, 1280)])
            pltpu.sync_copy(c_sh.at[pl.ds(off, 1280)],
                            cnt_h.at[c].at[pl.ds(off, 1280)])
            return _

        lax.fori_loop(0, olen // 1280, out, None)

    return pl.kernel(
        body,
        out_type=(
            jax.ShapeDtypeStruct((_NC, 2, e), jnp.float32),
            jax.ShapeDtypeStruct((_NC, e), jnp.float32),
        ),
        mesh=mesh,
        compiler_params=pltpu.CompilerParams(needs_layout_passes=False),
        scratch_types=[
            pltpu.VMEM((n,), jnp.int32),
            pltpu.VMEM((2 * na * na,), jnp.float32),
            pltpu.VMEM((2048,), jnp.int32),
            pltpu.VMEM((2048,), jnp.int32),
            pltpu.VMEM((8, 128), jnp.int32),
            pltpu.VMEM((8, 128), jnp.int32),
            pltpu.VMEM((1024,), jnp.float32),
            pltpu.VMEM((1024,), jnp.float32),
            pltpu.VMEM((128,), jnp.int32),
            pltpu.VMEM((128,), jnp.int32),
            pltpu.VMEM((8, 128), jnp.float32),
            pltpu.VMEM((8, 128), jnp.float32),
            pltpu.VMEM((128,), jnp.float32),
            pltpu.VMEM((2048,), jnp.float32),
            pltpu.VMEM_SHARED((ep,), jnp.int32),
            pltpu.VMEM_SHARED((ep,), jnp.int32),
            pltpu.VMEM_SHARED((ep,), jnp.float32),
            pltpu.VMEM_SHARED((ep,), jnp.float32),
            pltpu.VMEM_SHARED((ep,), jnp.float32),
        ],
    )(an, src_p, dst_p, lsrc_p, ldst_p, sp0, sp1, wflat)


def _phase2_sc(an, src, dst, sv, cp, n, na, e, qn, cn):
    sb = 10240
    qt = qn // _NS
    ct = cn // _NS
    mesh = plsc.VectorSubcoreMesh(core_axis_name="c", subcore_axis_name="s")

    def body(an_h, src_h, dst_h, sv_h, cp_h, q_h, cnt_h,
             an_v, srcbuf, dstbuf, vabuf, vbbuf, cabuf, cbbuf,
             qidx2d, cval2d, sidx2d, ones_v, zq, q_sp, cnt_sp):
        c = lax.axis_index("c")
        s = lax.axis_index("s")
        pltpu.sync_copy(an_h, an_v)

        def zz(k, _):
            zq[pl.ds(k * 16, 16)] = jnp.zeros((16,), jnp.float32)
            return _

        lax.fori_loop(0, 128, zz, None)

        def oo(k, _):
            ones_v[pl.ds(k * 16, 16)] = jnp.ones((16,), jnp.float32)
            return _

        lax.fori_loop(0, 8, oo, None)

        def zql(k, _):
            pltpu.sync_copy(zq, q_sp.at[pl.ds(s * qt + k * 2048, 2048)])
            return _

        lax.fori_loop(0, qt // 2048, zql, None)
        pltpu.sync_copy(zq.at[pl.ds(0, ct)], cnt_sp.at[pl.ds(s * ct, ct)])
        plsc.subcore_barrier()

        estart = s * sb
        nch = jnp.minimum(sb, e - estart) // 1280

        def chunk(ch, _):
            eb = estart + ch * 1280
            pltpu.sync_copy(src_h.at[pl.ds(eb, 1280)], srcbuf)
            pltpu.sync_copy(dst_h.at[pl.ds(eb, 1280)], dstbuf)
            pltpu.sync_copy(sv_h.at[0].at[c].at[pl.ds(eb, 1280)], vabuf)
            pltpu.sync_copy(sv_h.at[1].at[c].at[pl.ds(eb, 1280)], vbbuf)
            pltpu.sync_copy(cp_h.at[0].at[pl.ds(eb, 1280)], cabuf)
            pltpu.sync_copy(cp_h.at[1].at[pl.ds(eb, 1280)], cbbuf)

            def grp(g, _):
                sl = pl.ds(g * 16, 16)
                sv16 = srcbuf[sl]
                dv = dstbuf[sl]
                kd = plsc.load_gather(an_v, [dv])
                coef = (vabuf[sl] + vbbuf[sl]) / jnp.maximum(
                    cabuf[sl] + cbbuf[sl], 1.0)
                qi = sv16 * na + kd
                gd = g // 8
                off = (g % 8) * 16
                qidx2d[gd, pl.ds(off, 16)] = qi
                cval2d[gd, pl.ds(off, 16)] = coef
                sidx2d[gd, pl.ds(off, 16)] = sv16
                return _

            lax.fori_loop(0, 80, grp, None)

            def sc1(k, _):
                pltpu.sync_copy(cval2d.at[k], q_sp.at[qidx2d.at[k]], add=True)
                return _

            lax.fori_loop(0, 10, sc1, None)

            @pl.when(c == 0)
            def _counts():
                def sc2(k, _):
                    pltpu.sync_copy(ones_v, cnt_sp.at[sidx2d.at[k]], add=True)
                    return _

                lax.fori_loop(0, 10, sc2, None)

            return _

        lax.fori_loop(0, nch, chunk, None)
        plsc.subcore_barrier()

        def qo(k, _):
            off = s * qt + k * 2048
            pltpu.sync_copy(q_sp.at[pl.ds(off, 2048)],
                            q_h.at[c].at[pl.ds(off, 2048)])
            return _

        lax.fori_loop(0, qt // 2048, qo, None)

        @pl.when(c == 0)
        def _cout():
            pltpu.sync_copy(cnt_sp.at[pl.ds(s * ct, ct)],
                            cnt_h.at[pl.ds(s * ct, ct)])

    return pl.kernel(
        body,
        out_type=(
            jax.ShapeDtypeStruct((_NC, qn), jnp.float32),
            jax.ShapeDtypeStruct((cn,), jnp.float32),
        ),
        mesh=mesh,
        compiler_params=pltpu.CompilerParams(needs_layout_passes=False),
        scratch_types=[
            pltpu.VMEM((n,), jnp.int32),
            pltpu.VMEM((1280,), jnp.int32),
            pltpu.VMEM((1280,), jnp.int32),
            pltpu.VMEM((1280,), jnp.float32),
            pltpu.VMEM((1280,), jnp.float32),
            pltpu.VMEM((1280,), jnp.float32),
            pltpu.VMEM((1280,), jnp.float32),
            pltpu.VMEM((10, 128), jnp.int32),
            pltpu.VMEM((10, 128), jnp.float32),
            pltpu.VMEM((10, 128), jnp.int32),
            pltpu.VMEM((128,), jnp.float32),
            pltpu.VMEM((2048,), jnp.float32),
            pltpu.VMEM_SHARED((qn,), jnp.float32),
            pltpu.VMEM_SHARED((cn,), jnp.float32),
        ],
    )(an, src, dst, sv, cp)


def _final_tc(q0, q1, v0, v1, cnt2):
    n, na = q0.shape
    outf = v0.shape[1]
    bn = 1000

    def body(q0_ref, q1_ref, v0_ref, v1_ref, c_ref, o_ref):
        acc = jnp.dot(q0_ref[...], v0_ref[...],
                      preferred_element_type=jnp.float32)
        acc = acc + jnp.dot(q1_ref[...], v1_ref[...],
                            preferred_element_type=jnp.float32)
        o_ref[...] = acc / jnp.maximum(c_ref[...], 1.0)

    return pl.pallas_call(
        body,
        grid=(n // bn,),
        in_specs=[
            pl.BlockSpec((bn, na), lambda i: (i, 0)),
            pl.BlockSpec((bn, na), lambda i: (i, 0)),
            pl.BlockSpec((na, outf), lambda i: (0, 0)),
            pl.BlockSpec((na, outf), lambda i: (0, 0)),
            pl.BlockSpec((bn, 1), lambda i: (i, 0)),
        ],
        out_specs=pl.BlockSpec((bn, outf), lambda i: (i, 0)),
        out_shape=jax.ShapeDtypeStruct((n, outf), jnp.float32),
    )(q0, q1, v0, v1, cnt2)


def kernel(atomic_number, edge_index, lg_edge_index, h, dnr,
           key_embedding, value_table, a, b, c, d):
    n = atomic_number.shape[0]
    e = edge_index.shape[1]
    t = lg_edge_index.shape[1]
    heads = a.shape[0]
    na = key_embedding.shape[0]
    hid = key_embedding.shape[1] // heads
    outf = value_table.shape[1] // heads

    ep = _round_up(e, _NS * 2048)
    tp = _round_up(t, _NC * _NS * 1024)
    qn = _round_up(n * na, _NS * 2048)
    cn = _round_up(n, _NS * 128)

    an32 = atomic_number.astype(jnp.int32)
    src = edge_index[0].astype(jnp.int32)
    dst = edge_index[1].astype(jnp.int32)
    lsrc = lg_edge_index[0].astype(jnp.int32)
    ldst = lg_edge_index[1].astype(jnp.int32)
    src_p = jnp.pad(src, (0, ep - e))
    dst_p = jnp.pad(dst, (0, ep - e))
    lsrc_p = jnp.pad(lsrc, (0, tp - t), constant_values=e)
    ldst_p = jnp.pad(ldst, (0, tp - t))

    h2 = h.reshape(t // 128, 128)
    dnr2 = dnr.reshape(t // 128, 128)
    kh = key_embedding.reshape(na, hid, heads).transpose(2, 0, 1)
    scal = jnp.stack([a, b % jnp.float32(np.pi), c, d])

    sp, w = _prep_tc(h2, dnr2, kh, scal)
    spf = sp.reshape(2, t)
    sp0 = jnp.pad(spf[0], (0, tp - t))
    sp1 = jnp.pad(spf[1], (0, tp - t))
    wflat = w.reshape(2 * na * na)

    sv, cp = _phase1_sc(an32, src_p, dst_p, lsrc_p, ldst_p, sp0, sp1, wflat,
                        n, na, e, ep, tp)
    q_out, cnt_out = _phase2_sc(an32, src, dst, sv, cp, n, na, e, qn, cn)

    q = q_out[:, :n * na].reshape(2, n, na)
    v = value_table.reshape(na, outf, heads)
    cnt2 = cnt_out[:n].reshape(n, 1)
    return _final_tc(q[0], q[1], v[:, :, 0], v[:, :, 1], cnt2)


# speedup vs baseline: 159.9431x; 1.2934x over previous
import numpy as np
import jax
import jax.numpy as jnp
from jax import lax
from jax.experimental import pallas as pl
from jax.experimental.pallas import tpu as pltpu
from jax.experimental.pallas import tpu_sc as plsc

_EPS = 1e-3
_NC, _NS = 2, 16


def _round_up(x, m):
    return (x + m - 1) // m * m


def _prep_tc(h2, dnr2, kh, scal):
    tch = h2.shape[0]
    na, hid = kh.shape[1], kh.shape[2]

    def body(h_ref, d_ref, k_ref, s_ref, sp_ref, w_ref):
        x = jnp.clip(h_ref[...], -_EPS, _EPS)
        theta = jnp.float32(np.pi / 2) - x - (x * x * x) * jnp.float32(1.0 / 6.0)
        d2 = d_ref[...] * d_ref[...]
        for hd in range(2):
            av = s_ref[0, hd]
            bv = s_ref[1, hd]
            cv = s_ref[2, hd]
            dv = s_ref[3, hd]
            ang = ((jnp.cos(av * theta + bv) + 1.0) * 0.5) ** cv
            rad = jnp.exp(-dv * d2)
            sp_ref[hd] = ang * rad
            k = k_ref[hd]
            w_ref[hd] = lax.dot_general(
                k, k, (((1,), (1,)), ((), ())),
                preferred_element_type=jnp.float32)

    return pl.pallas_call(
        body,
        in_specs=[
            pl.BlockSpec(memory_space=pltpu.VMEM),
            pl.BlockSpec(memory_space=pltpu.VMEM),
            pl.BlockSpec(memory_space=pltpu.VMEM),
            pl.BlockSpec(memory_space=pltpu.SMEM),
        ],
        out_shape=(
            jax.ShapeDtypeStruct((2, tch, 128), jnp.float32),
            jax.ShapeDtypeStruct((2, na, na), jnp.float32),
        ),
    )(h2, dnr2, kh, scal)


def _phase1_sc(an, src_p, dst_p, lsrc2, ldst2, sp02, sp12, wflat,
               n, na, e, ep, tp):
    ept = ep // _NS
    tt = tp // (_NC * _NS)
    nch = tt // 1024
    mesh = plsc.VectorSubcoreMesh(core_axis_name="c", subcore_axis_name="s")

    def body(an_h, src_h, dst_h, lsrc_h, ldst_h, sp0_h, sp1_h, w_h,
             sv_h, cnt_h,
             an_v, w_v, srcbuf, idsbuf,
             lsA, lsB, ldA, ldB, s0A, s0B, s1A, s1B,
             kiA, kiB, kjA, kjB, v0A, v0B, v1A, v1B,
             ones_v, zbuf, lsemA, lsemB, gsemA, gsemB, ssemA, ssemB,
             ks_sh, kd_sh, s0_sh, s1_sh, c_sh):
        c = lax.axis_index("c")
        s = lax.axis_index("s")
        pltpu.sync_copy(an_h, an_v)
        pltpu.sync_copy(w_h, w_v)

        def fill_ids(eh, sh):
            def blk(bi, _):
                ebase = s * ept + bi * 2048
                pltpu.sync_copy(eh.at[pl.ds(ebase, 2048)], srcbuf)

                def gg(g, _):
                    idx = srcbuf[pl.ds(g * 16, 16)]
                    idsbuf[pl.ds(g * 16, 16)] = plsc.load_gather(an_v, [idx])
                    return _

                lax.fori_loop(0, 128, gg, None)
                pltpu.sync_copy(idsbuf, sh.at[pl.ds(ebase, 2048)])
                return _

            lax.fori_loop(0, ept // 2048, blk, None)

        fill_ids(src_h, ks_sh)
        fill_ids(dst_h, kd_sh)

        def zz(k, _):
            zbuf[pl.ds(k * 16, 16)] = jnp.zeros((16,), jnp.float32)
            return _

        lax.fori_loop(0, 128, zz, None)

        def oo(k, _):
            ones_v[pl.ds(k * 16, 16)] = jnp.ones((16,), jnp.float32)
            return _

        lax.fori_loop(0, 8, oo, None)

        def z2(k, _):
            off = s * (ep // _NS) + k * 2048
            pltpu.sync_copy(zbuf, s0_sh.at[pl.ds(off, 2048)])
            pltpu.sync_copy(zbuf, s1_sh.at[pl.ds(off, 2048)])
            pltpu.sync_copy(zbuf, c_sh.at[pl.ds(off, 2048)])
            return _

        lax.fori_loop(0, ep // _NS // 2048, z2, None)
        plsc.subcore_barrier()

        ls = [lsA, lsB]
        ld = [ldA, ldB]
        s0b = [s0A, s0B]
        s1b = [s1A, s1B]
        ki = [kiA, kiB]
        kj = [kjA, kjB]
        v0b = [v0A, v0B]
        v1b = [v1A, v1B]
        lsems = [lsemA, lsemB]
        gsems = [gsemA, gsemB]
        ssems = [ssemA, ssemB]
        rowbase = c * (tp // 2 // 128) + s * (tt // 128)

        def issue_loads(ch):
            p = ch % 2
            rb = pl.multiple_of(rowbase + ch * 8, 8)
            return [
                pltpu.async_copy(lsrc_h.at[pl.ds(rb, 8)], ls[p], lsems[p]),
                pltpu.async_copy(ldst_h.at[pl.ds(rb, 8)], ld[p], lsems[p]),
                pltpu.async_copy(sp0_h.at[pl.ds(rb, 8)], s0b[p], lsems[p]),
                pltpu.async_copy(sp1_h.at[pl.ds(rb, 8)], s1b[p], lsems[p]),
            ]

        def issue_gathers(p, si):
            gp = si % 2
            return [
                pltpu.async_copy(ks_sh.at[ls[p].at[si]], ki[gp], gsems[gp]),
                pltpu.async_copy(kd_sh.at[ld[p].at[si]], kj[gp], gsems[gp]),
            ]

        loads = {0: issue_loads(0)}
        scats = {}
        for ch in range(nch):
            p = ch % 2
            if ch >= 1:
                for dd in scats[ch - 1]:
                    dd.wait()
            if ch + 1 < nch:
                loads[ch + 1] = issue_loads(ch + 1)
            for dd in loads[ch]:
                dd.wait()
            gcur = issue_gathers(p, 0)
            pend = []
            for si in range(8):
                gp = si % 2
                if si < 7:
                    gnext = issue_gathers(p, si + 1)
                for dd in gcur:
                    dd.wait()

                def grp(g, _, si=si, gp=gp, p=p):
                    i = ki[gp][pl.ds(g * 16, 16)]
                    j = kj[gp][pl.ds(g * 16, 16)]
                    fidx = i * na + j
                    w0 = plsc.load_gather(w_v, [fidx])
                    w1 = plsc.load_gather(w_v, [fidx + na * na])
                    v0b[p][si, pl.ds(g * 16, 16)] = (
                        w0 * s0b[p][si, pl.ds(g * 16, 16)])
                    v1b[p][si, pl.ds(g * 16, 16)] = (
                        w1 * s1b[p][si, pl.ds(g * 16, 16)])
                    return _

                lax.fori_loop(0, 8, grp, None)
                pend.append(pltpu.async_copy(
                    v0b[p].at[si], s0_sh.at[ls[p].at[si]], ssems[p],
                    add=True))
                pend.append(pltpu.async_copy(
                    v1b[p].at[si], s1_sh.at[ls[p].at[si]], ssems[p],
                    add=True))
                pend.append(pltpu.async_copy(
                    ones_v, c_sh.at[ls[p].at[si]], ssems[p], add=True))
                gcur = gnext
            scats[ch] = pend
        for dd in scats[nch - 1]:
            dd.wait()
        plsc.subcore_barrier()

        ostart = s * (ep // _NS)

        def out(k, _):
            off = ostart + k * 1280
            pltpu.sync_copy(s0_sh.at[pl.ds(off, 1280)],
                            sv_h.at[c].at[0].at[pl.ds(off, 1280)])
            pltpu.sync_copy(s1_sh.at[pl.ds(off, 1280)],
                            sv_h.at[c].at[1].at[pl.ds(off, 1280)])
            pltpu.sync_copy(c_sh.at[pl.ds(off, 1280)],
                            cnt_h.at[c].at[pl.ds(off, 1280)])
            return _

        lax.fori_loop(0, ep // _NS // 1280, out, None)

    return pl.kernel(
        body,
        out_type=(
            jax.ShapeDtypeStruct((_NC, 2, ep), jnp.float32),
            jax.ShapeDtypeStruct((_NC, ep), jnp.float32),
        ),
        mesh=mesh,
        compiler_params=pltpu.CompilerParams(needs_layout_passes=False),
        scratch_types=[
            pltpu.VMEM((n,), jnp.int32),
            pltpu.VMEM((2 * na * na,), jnp.float32),
            pltpu.VMEM((2048,), jnp.int32),
            pltpu.VMEM((2048,), jnp.int32),
            pltpu.VMEM((8, 128), jnp.int32),
            pltpu.VMEM((8, 128), jnp.int32),
            pltpu.VMEM((8, 128), jnp.int32),
            pltpu.VMEM((8, 128), jnp.int32),
            pltpu.VMEM((8, 128), jnp.float32),
            pltpu.VMEM((8, 128), jnp.float32),
            pltpu.VMEM((8, 128), jnp.float32),
            pltpu.VMEM((8, 128), jnp.float32),
            pltpu.VMEM((128,), jnp.int32),
            pltpu.VMEM((128,), jnp.int32),
            pltpu.VMEM((128,), jnp.int32),
            pltpu.VMEM((128,), jnp.int32),
            pltpu.VMEM((8, 128), jnp.float32),
            pltpu.VMEM((8, 128), jnp.float32),
            pltpu.VMEM((8, 128), jnp.float32),
            pltpu.VMEM((8, 128), jnp.float32),
            pltpu.VMEM((128,), jnp.float32),
            pltpu.VMEM((2048,), jnp.float32),
            pltpu.SemaphoreType.DMA,
            pltpu.SemaphoreType.DMA,
            pltpu.SemaphoreType.DMA,
            pltpu.SemaphoreType.DMA,
            pltpu.SemaphoreType.DMA,
            pltpu.SemaphoreType.DMA,
            pltpu.VMEM_SHARED((ep,), jnp.int32),
            pltpu.VMEM_SHARED((ep,), jnp.int32),
            pltpu.VMEM_SHARED((ep,), jnp.float32),
            pltpu.VMEM_SHARED((ep,), jnp.float32),
            pltpu.VMEM_SHARED((ep,), jnp.float32),
        ],
    )(an, src_p, dst_p, lsrc2, ldst2, sp02, sp12, wflat)


def _phase2_sc(an, src2, dst2, sv, cp, n, na, ep, qn, cn):
    et = ep // _NS
    nch = et // 2048
    qt = qn // _NS
    ct = cn // _NS
    mesh = plsc.VectorSubcoreMesh(core_axis_name="c", subcore_axis_name="s")

    def body(an_h, src_h, dst_h, sv_h, cp_h, q_h, cnt_h,
             an_v, srA, srB, dsA, dsB,
             vaA, vaB, vbA, vbB, caA, caB, cbA, cbB,
             qiA, qiB, cvA, cvB,
             ones_v, zq, lsemA, lsemB, ssemA, ssemB, q_sp, cnt_sp):
        c = lax.axis_index("c")
        s = lax.axis_index("s")
        pltpu.sync_copy(an_h, an_v)

        def zz(k, _):
            zq[pl.ds(k * 16, 16)] = jnp.zeros((16,), jnp.float32)
            return _

        lax.fori_loop(0, 128, zz, None)

        def oo(k, _):
            ones_v[pl.ds(k * 16, 16)] = jnp.ones((16,), jnp.float32)
            return _

        lax.fori_loop(0, 8, oo, None)

        def zql(k, _):
            pltpu.sync_copy(zq, q_sp.at[pl.ds(s * qt + k * 2048, 2048)])
            return _

        lax.fori_loop(0, qt // 2048, zql, None)
        pltpu.sync_copy(zq.at[pl.ds(0, ct)], cnt_sp.at[pl.ds(s * ct, ct)])
        plsc.subcore_barrier()

        sr = [srA, srB]
        ds2 = [dsA, dsB]
        va = [vaA, vaB]
        vb = [vbA, vbB]
        ca = [caA, caB]
        cb = [cbA, cbB]
        qi2 = [qiA, qiB]
        cv2 = [cvA, cvB]
        lsems = [lsemA, lsemB]
        ssems = [ssemA, ssemB]
        estart = s * et
        rstart = s * (et // 128)

        def issue_loads(ch):
            p = ch % 2
            eb = pl.multiple_of(estart + ch * 2048, 2048)
            rb = pl.multiple_of(rstart + ch * 16, 16)
            return [
                pltpu.async_copy(src_h.at[pl.ds(rb, 16)], sr[p], lsems[p]),
                pltpu.async_copy(dst_h.at[pl.ds(rb, 16)], ds2[p], lsems[p]),
                pltpu.async_copy(sv_h.at[0].at[c].at[pl.ds(eb, 2048)],
                                 va[p], lsems[p]),
                pltpu.async_copy(sv_h.at[1].at[c].at[pl.ds(eb, 2048)],
                                 vb[p], lsems[p]),
                pltpu.async_copy(cp_h.at[0].at[pl.ds(eb, 2048)],
                                 ca[p], lsems[p]),
                pltpu.async_copy(cp_h.at[1].at[pl.ds(eb, 2048)],
                                 cb[p], lsems[p]),
            ]

        loads = {0: issue_loads(0)}
        scats = {}
        for ch in range(nch):
            p = ch % 2
            if ch >= 1:
                for dd in scats[ch - 1]:
                    dd.wait()
            if ch + 1 < nch:
                loads[ch + 1] = issue_loads(ch + 1)
            for dd in loads[ch]:
                dd.wait()

            def grp(g, _, p=p):
                gd = g // 8
                off = (g % 8) * 16
                sv16 = sr[p][gd, pl.ds(off, 16)]
                dv = ds2[p][gd, pl.ds(off, 16)]
                kd = plsc.load_gather(an_v, [dv])
                sl = pl.ds(g * 16, 16)
                coef = (va[p][sl] + vb[p][sl]) / jnp.maximum(
                    ca[p][sl] + cb[p][sl], 1.0)
                qi2[p][gd, pl.ds(off, 16)] = sv16 * na + kd
                cv2[p][gd, pl.ds(off, 16)] = coef
                return _

            lax.fori_loop(0, 128, grp, None)
            pend = []
            for k in range(16):
                pend.append(pltpu.async_copy(
                    cv2[p].at[k], q_sp.at[qi2[p].at[k]], ssems[p], add=True))
                pend.append(pltpu.async_copy(
                    ones_v, cnt_sp.at[sr[p].at[k]], ssems[p], add=True))
            scats[ch] = pend

        for dd in scats[nch - 1]:
            dd.wait()
        plsc.subcore_barrier()

        def qo(k, _):
            off = s * qt + k * 2048
            pltpu.sync_copy(q_sp.at[pl.ds(off, 2048)],
                            q_h.at[c].at[pl.ds(off, 2048)])
            return _

        lax.fori_loop(0, qt // 2048, qo, None)

        @pl.when(c == 0)
        def _cout():
            pltpu.sync_copy(cnt_sp.at[pl.ds(s * ct, ct)],
                            cnt_h.at[pl.ds(s * ct, ct)])

    return pl.kernel(
        body,
        out_type=(
            jax.ShapeDtypeStruct((_NC, qn), jnp.float32),
            jax.ShapeDtypeStruct((cn,), jnp.float32),
        ),
        mesh=mesh,
        compiler_params=pltpu.CompilerParams(needs_layout_passes=False),
        scratch_types=[
            pltpu.VMEM((n,), jnp.int32),
            pltpu.VMEM((16, 128), jnp.int32),
            pltpu.VMEM((16, 128), jnp.int32),
            pltpu.VMEM((16, 128), jnp.int32),
            pltpu.VMEM((16, 128), jnp.int32),
            pltpu.VMEM((2048,), jnp.float32),
            pltpu.VMEM((2048,), jnp.float32),
            pltpu.VMEM((2048,), jnp.float32),
            pltpu.VMEM((2048,), jnp.float32),
            pltpu.VMEM((2048,), jnp.float32),
            pltpu.VMEM((2048,), jnp.float32),
            pltpu.VMEM((2048,), jnp.float32),
            pltpu.VMEM((2048,), jnp.float32),
            pltpu.VMEM((16, 128), jnp.int32),
            pltpu.VMEM((16, 128), jnp.int32),
            pltpu.VMEM((16, 128), jnp.float32),
            pltpu.VMEM((16, 128), jnp.float32),
            pltpu.VMEM((128,), jnp.float32),
            pltpu.VMEM((2048,), jnp.float32),
            pltpu.SemaphoreType.DMA,
            pltpu.SemaphoreType.DMA,
            pltpu.SemaphoreType.DMA,
            pltpu.SemaphoreType.DMA,
            pltpu.VMEM_SHARED((qn,), jnp.float32),
            pltpu.VMEM_SHARED((cn,), jnp.float32),
        ],
    )(an, src2, dst2, sv, cp)


def _final_tc(q0, q1, v0, v1, cnt2):
    n, na = q0.shape
    outf = v0.shape[1]
    bn = 1000

    def body(q0_ref, q1_ref, v0_ref, v1_ref, c_ref, o_ref):
        acc = jnp.dot(q0_ref[...], v0_ref[...],
                      preferred_element_type=jnp.float32)
        acc = acc + jnp.dot(q1_ref[...], v1_ref[...],
                            preferred_element_type=jnp.float32)
        o_ref[...] = acc / jnp.maximum(c_ref[...], 1.0)

    return pl.pallas_call(
        body,
        grid=(n // bn,),
        in_specs=[
            pl.BlockSpec((bn, na), lambda i: (i, 0)),
            pl.BlockSpec((bn, na), lambda i: (i, 0)),
            pl.BlockSpec((na, outf), lambda i: (0, 0)),
            pl.BlockSpec((na, outf), lambda i: (0, 0)),
            pl.BlockSpec((bn, 1), lambda i: (i, 0)),
        ],
        out_specs=pl.BlockSpec((bn, outf), lambda i: (i, 0)),
        out_shape=jax.ShapeDtypeStruct((n, outf), jnp.float32),
    )(q0, q1, v0, v1, cnt2)


def kernel(atomic_number, edge_index, lg_edge_index, h, dnr,
           key_embedding, value_table, a, b, c, d):
    n = atomic_number.shape[0]
    e = edge_index.shape[1]
    t = lg_edge_index.shape[1]
    heads = a.shape[0]
    na = key_embedding.shape[0]
    hid = key_embedding.shape[1] // heads
    outf = value_table.shape[1] // heads

    ep = _round_up(e, _NS * 2048)
    tp = _round_up(t, _NC * _NS * 1024)
    qn = _round_up(n * na + na, _NS * 2048)
    cn = _round_up(n + 1, _NS * 128)

    an32 = atomic_number.astype(jnp.int32)
    src = edge_index[0].astype(jnp.int32)
    dst = edge_index[1].astype(jnp.int32)
    lsrc = lg_edge_index[0].astype(jnp.int32)
    ldst = lg_edge_index[1].astype(jnp.int32)
    src_p = jnp.pad(src, (0, ep - e))
    dst_p = jnp.pad(dst, (0, ep - e))
    src_p2 = jnp.pad(src, (0, ep - e), constant_values=n).reshape(
        ep // 128, 128)
    dst_p2 = dst_p.reshape(ep // 128, 128)
    lsrc2 = jnp.pad(lsrc, (0, tp - t), constant_values=e).reshape(
        tp // 128, 128)
    ldst2 = jnp.pad(ldst, (0, tp - t)).reshape(tp // 128, 128)

    h2 = h.reshape(t // 128, 128)
    dnr2 = dnr.reshape(t // 128, 128)
    kh = key_embedding.reshape(na, hid, heads).transpose(2, 0, 1)
    scal = jnp.stack([a, b % jnp.float32(np.pi), c, d])

    sp, w = _prep_tc(h2, dnr2, kh, scal)
    spf = sp.reshape(2, t)
    sp02 = jnp.pad(spf[0], (0, tp - t)).reshape(tp // 128, 128)
    sp12 = jnp.pad(spf[1], (0, tp - t)).reshape(tp // 128, 128)
    wflat = w.reshape(2 * na * na)

    sv, cp = _phase1_sc(an32, src_p, dst_p, lsrc2, ldst2, sp02, sp12, wflat,
                        n, na, e, ep, tp)
    q_out, cnt_out = _phase2_sc(an32, src_p2, dst_p2, sv, cp,
                                n, na, ep, qn, cn)

    q = q_out[:, :n * na].reshape(2, n, na)
    v = value_table.reshape(na, outf, heads)
    cnt2 = cnt_out[:n].reshape(n, 1)
    return _final_tc(q[0], q[1], v[:, :, 0], v[:, :, 1], cnt2)


# speedup vs baseline: 228.5911x; 1.4292x over previous
import numpy as np
import jax
import jax.numpy as jnp
from jax import lax
from jax.experimental import pallas as pl
from jax.experimental.pallas import tpu as pltpu
from jax.experimental.pallas import tpu_sc as plsc

_EPS = 1e-3
_NC, _NS = 2, 16


def _round_up(x, m):
    return (x + m - 1) // m * m


def _prep_tc(h2, dnr2, kh, scal):
    tch = h2.shape[0]
    na, hid = kh.shape[1], kh.shape[2]

    def body(h_ref, d_ref, k_ref, s_ref, sp_ref, w_ref):
        x = jnp.clip(h_ref[...], -_EPS, _EPS)
        theta = jnp.float32(np.pi / 2) - x - (x * x * x) * jnp.float32(1.0 / 6.0)
        d2 = d_ref[...] * d_ref[...]
        for hd in range(2):
            av = s_ref[0, hd]
            bv = s_ref[1, hd]
            cv = s_ref[2, hd]
            dv = s_ref[3, hd]
            ang = ((jnp.cos(av * theta + bv) + 1.0) * 0.5) ** cv
            rad = jnp.exp(-dv * d2)
            sp_ref[hd] = ang * rad
            k = k_ref[hd]
            w_ref[hd] = lax.dot_general(
                k, k, (((1,), (1,)), ((), ())),
                preferred_element_type=jnp.float32)

    return pl.pallas_call(
        body,
        in_specs=[
            pl.BlockSpec(memory_space=pltpu.VMEM),
            pl.BlockSpec(memory_space=pltpu.VMEM),
            pl.BlockSpec(memory_space=pltpu.VMEM),
            pl.BlockSpec(memory_space=pltpu.SMEM),
        ],
        out_shape=(
            jax.ShapeDtypeStruct((2, tch, 128), jnp.float32),
            jax.ShapeDtypeStruct((2, na, na), jnp.float32),
        ),
    )(h2, dnr2, kh, scal)


def _phase1_sc(an, src_p, dst_p, lsrc2, ldst2, sp02, sp12, wflat,
               n, na, e, ep, tp):
    ept = ep // _NS
    tt = tp // (_NC * _NS)
    nch = tt // 1024
    mesh = plsc.VectorSubcoreMesh(core_axis_name="c", subcore_axis_name="s")

    def body(an_h, src_h, dst_h, lsrc_h, ldst_h, sp0_h, sp1_h, w_h,
             sv_h, cnt_h,
             an_v, w_v, srcbufA, srcbufB, idsbufA, idsbufB,
             lsA, lsB, ldA, ldB, s0A, s0B, s1A, s1B,
             kiA, kiB, kjA, kjB, v0A, v0B, v1A, v1B,
             ones_v, zbuf, lsemA, lsemB, gsemA, gsemB, ssemA, ssemB,
             ks_sh, kd_sh, s0_sh, s1_sh, c_sh):
        c = lax.axis_index("c")
        s = lax.axis_index("s")
        lsems = [lsemA, lsemB]
        ssems = [ssemA, ssemB]
        pltpu.sync_copy(an_h, an_v)
        pltpu.sync_copy(w_h, w_v)

        sbufs = [srcbufA, srcbufB]
        ibufs = [idsbufA, idsbufB]
        steps = ([(src_h, ks_sh, bi) for bi in range(ept // 2048)]
                 + [(dst_h, kd_sh, bi) for bi in range(ept // 2048)])
        nst = len(steps)

        def s1_load(i):
            eh, _, bi = steps[i]
            ebase = pl.multiple_of(s * ept + bi * 2048, 2048)
            return pltpu.async_copy(eh.at[pl.ds(ebase, 2048)],
                                    sbufs[i % 2], lsems[i % 2])

        ldd = {0: s1_load(0)}
        std = {}
        for i in range(nst):
            p = i % 2
            if i >= 2:
                std[i - 2].wait()
            if i + 1 < nst:
                ldd[i + 1] = s1_load(i + 1)
            ldd[i].wait()

            def gg(g, _, p=p):
                for u in range(4):
                    sl = pl.ds((g * 4 + u) * 16, 16)
                    ibufs[p][sl] = plsc.load_gather(an_v, [sbufs[p][sl]])
                return _

            lax.fori_loop(0, 32, gg, None)
            _, sh, bi = steps[i]
            ebase = pl.multiple_of(s * ept + bi * 2048, 2048)
            std[i] = pltpu.async_copy(ibufs[p], sh.at[pl.ds(ebase, 2048)],
                                      ssems[p])
        std[nst - 2].wait()
        std[nst - 1].wait()

        def zz(k, _):
            zbuf[pl.ds(k * 16, 16)] = jnp.zeros((16,), jnp.float32)
            return _

        lax.fori_loop(0, 128, zz, None)

        def oo(k, _):
            ones_v[pl.ds(k * 16, 16)] = jnp.ones((16,), jnp.float32)
            return _

        lax.fori_loop(0, 8, oo, None)

        def z2(k, _):
            off = s * (ep // _NS) + k * 2048
            pltpu.sync_copy(zbuf, s0_sh.at[pl.ds(off, 2048)])
            pltpu.sync_copy(zbuf, s1_sh.at[pl.ds(off, 2048)])
            pltpu.sync_copy(zbuf, c_sh.at[pl.ds(off, 2048)])
            return _

        lax.fori_loop(0, ep // _NS // 2048, z2, None)
        plsc.subcore_barrier()

        ls = [lsA, lsB]
        ld = [ldA, ldB]
        s0b = [s0A, s0B]
        s1b = [s1A, s1B]
        ki = [kiA, kiB]
        kj = [kjA, kjB]
        v0b = [v0A, v0B]
        v1b = [v1A, v1B]
        lsems = [lsemA, lsemB]
        gsems = [gsemA, gsemB]
        ssems = [ssemA, ssemB]
        rowbase = c * (tp // 2 // 128) + s * (tt // 128)

        def issue_loads(ch):
            p = ch % 2
            rb = pl.multiple_of(rowbase + ch * 8, 8)
            return [
                pltpu.async_copy(lsrc_h.at[pl.ds(rb, 8)], ls[p], lsems[p]),
                pltpu.async_copy(ldst_h.at[pl.ds(rb, 8)], ld[p], lsems[p]),
                pltpu.async_copy(sp0_h.at[pl.ds(rb, 8)], s0b[p], lsems[p]),
                pltpu.async_copy(sp1_h.at[pl.ds(rb, 8)], s1b[p], lsems[p]),
            ]

        def issue_gathers(p, si):
            gp = si % 2
            return [
                pltpu.async_copy(ks_sh.at[ls[p].at[si]], ki[gp], gsems[gp]),
                pltpu.async_copy(kd_sh.at[ld[p].at[si]], kj[gp], gsems[gp]),
            ]

        loads = {0: issue_loads(0)}
        scats = {}
        for ch in range(nch):
            p = ch % 2
            if ch >= 1:
                for dd in scats[ch - 1]:
                    dd.wait()
            if ch + 1 < nch:
                loads[ch + 1] = issue_loads(ch + 1)
            for dd in loads[ch]:
                dd.wait()
            gcur = issue_gathers(p, 0)
            pend = []
            for si in range(8):
                gp = si % 2
                if si < 7:
                    gnext = issue_gathers(p, si + 1)
                for dd in gcur:
                    dd.wait()

                def grp(g, _, si=si, gp=gp, p=p):
                    i = ki[gp][pl.ds(g * 16, 16)]
                    j = kj[gp][pl.ds(g * 16, 16)]
                    fidx = i * na + j
                    w0 = plsc.load_gather(w_v, [fidx])
                    w1 = plsc.load_gather(w_v, [fidx + na * na])
                    v0b[p][si, pl.ds(g * 16, 16)] = (
                        w0 * s0b[p][si, pl.ds(g * 16, 16)])
                    v1b[p][si, pl.ds(g * 16, 16)] = (
                        w1 * s1b[p][si, pl.ds(g * 16, 16)])
                    return _

                lax.fori_loop(0, 8, grp, None)
                pend.append(pltpu.async_copy(
                    v0b[p].at[si], s0_sh.at[ls[p].at[si]], ssems[p],
                    add=True))
                pend.append(pltpu.async_copy(
                    v1b[p].at[si], s1_sh.at[ls[p].at[si]], ssems[p],
                    add=True))
                pend.append(pltpu.async_copy(
                    ones_v, c_sh.at[ls[p].at[si]], ssems[p], add=True))
                gcur = gnext
            scats[ch] = pend
        for dd in scats[nch - 1]:
            dd.wait()
        plsc.subcore_barrier()

        ostart = s * (ep // _NS)

        def out(k, _):
            off = ostart + k * 1280
            pltpu.sync_copy(s0_sh.at[pl.ds(off, 1280)],
                            sv_h.at[c].at[0].at[pl.ds(off, 1280)])
            pltpu.sync_copy(s1_sh.at[pl.ds(off, 1280)],
                            sv_h.at[c].at[1].at[pl.ds(off, 1280)])
            pltpu.sync_copy(c_sh.at[pl.ds(off, 1280)],
                            cnt_h.at[c].at[pl.ds(off, 1280)])
            return _

        lax.fori_loop(0, ep // _NS // 1280, out, None)

    return pl.kernel(
        body,
        out_type=(
            jax.ShapeDtypeStruct((_NC, 2, ep), jnp.float32),
            jax.ShapeDtypeStruct((_NC, ep), jnp.float32),
        ),
        mesh=mesh,
        compiler_params=pltpu.CompilerParams(needs_layout_passes=False),
        scratch_types=[
            pltpu.VMEM((n,), jnp.int32),
            pltpu.VMEM((2 * na * na,), jnp.float32),
            pltpu.VMEM((2048,), jnp.int32),
            pltpu.VMEM((2048,), jnp.int32),
            pltpu.VMEM((2048,), jnp.int32),
            pltpu.VMEM((2048,), jnp.int32),
            pltpu.VMEM((8, 128), jnp.int32),
            pltpu.VMEM((8, 128), jnp.int32),
            pltpu.VMEM((8, 128), jnp.int32),
            pltpu.VMEM((8, 128), jnp.int32),
            pltpu.VMEM((8, 128), jnp.float32),
            pltpu.VMEM((8, 128), jnp.float32),
            pltpu.VMEM((8, 128), jnp.float32),
            pltpu.VMEM((8, 128), jnp.float32),
            pltpu.VMEM((128,), jnp.int32),
            pltpu.VMEM((128,), jnp.int32),
            pltpu.VMEM((128,), jnp.int32),
            pltpu.VMEM((128,), jnp.int32),
            pltpu.VMEM((8, 128), jnp.float32),
            pltpu.VMEM((8, 128), jnp.float32),
            pltpu.VMEM((8, 128), jnp.float32),
            pltpu.VMEM((8, 128), jnp.float32),
            pltpu.VMEM((128,), jnp.float32),
            pltpu.VMEM((2048,), jnp.float32),
            pltpu.SemaphoreType.DMA,
            pltpu.SemaphoreType.DMA,
            pltpu.SemaphoreType.DMA,
            pltpu.SemaphoreType.DMA,
            pltpu.SemaphoreType.DMA,
            pltpu.SemaphoreType.DMA,
            pltpu.VMEM_SHARED((ep,), jnp.int32),
            pltpu.VMEM_SHARED((ep,), jnp.int32),
            pltpu.VMEM_SHARED((ep,), jnp.float32),
            pltpu.VMEM_SHARED((ep,), jnp.float32),
            pltpu.VMEM_SHARED((ep,), jnp.float32),
        ],
    )(an, src_p, dst_p, lsrc2, ldst2, sp02, sp12, wflat)


def _phase2_sc(an, src2, dst2, sv, cp, n, np2, nap, ep):
    qn = nap * np2
    et = ep // _NS
    nch = et // 2048
    qt = qn // _NS
    ct = np2 // _NS
    mesh = plsc.VectorSubcoreMesh(core_axis_name="c", subcore_axis_name="s")

    def body(an_h, src_h, dst_h, sv_h, cp_h, q_h, cnt_h,
             an_v, srA, srB, dsA, dsB,
             vaA, vaB, vbA, vbB, caA, caB, cbA, cbB,
             qiA, qiB, cvA, cvB,
             ones_v, zq, lsemA, lsemB, ssemA, ssemB, q_sp, cnt_sp):
        c = lax.axis_index("c")
        s = lax.axis_index("s")
        pltpu.sync_copy(an_h, an_v)

        def zz(k, _):
            zq[pl.ds(k * 16, 16)] = jnp.zeros((16,), jnp.float32)
            return _

        lax.fori_loop(0, 160, zz, None)

        def oo(k, _):
            ones_v[pl.ds(k * 16, 16)] = jnp.ones((16,), jnp.float32)
            return _

        lax.fori_loop(0, 8, oo, None)

        def zql(k, _):
            pltpu.sync_copy(zq, q_sp.at[pl.ds(s * qt + k * 2560, 2560)])
            return _

        lax.fori_loop(0, qt // 2560, zql, None)
        pltpu.sync_copy(zq.at[pl.ds(0, ct)], cnt_sp.at[pl.ds(s * ct, ct)])
        plsc.subcore_barrier()

        sr = [srA, srB]
        ds2 = [dsA, dsB]
        va = [vaA, vaB]
        vb = [vbA, vbB]
        ca = [caA, caB]
        cb = [cbA, cbB]
        qi2 = [qiA, qiB]
        cv2 = [cvA, cvB]
        lsems = [lsemA, lsemB]
        ssems = [ssemA, ssemB]
        estart = s * et
        rstart = s * (et // 128)

        def issue_loads(ch):
            p = ch % 2
            eb = pl.multiple_of(estart + ch * 2048, 2048)
            rb = pl.multiple_of(rstart + ch * 16, 16)
            return [
                pltpu.async_copy(src_h.at[pl.ds(rb, 16)], sr[p], lsems[p]),
                pltpu.async_copy(dst_h.at[pl.ds(rb, 16)], ds2[p], lsems[p]),
                pltpu.async_copy(sv_h.at[0].at[c].at[pl.ds(eb, 2048)],
                                 va[p], lsems[p]),
                pltpu.async_copy(sv_h.at[1].at[c].at[pl.ds(eb, 2048)],
                                 vb[p], lsems[p]),
                pltpu.async_copy(cp_h.at[0].at[pl.ds(eb, 2048)],
                                 ca[p], lsems[p]),
                pltpu.async_copy(cp_h.at[1].at[pl.ds(eb, 2048)],
                                 cb[p], lsems[p]),
            ]

        loads = {0: issue_loads(0)}
        scats = {}
        for ch in range(nch):
            p = ch % 2
            if ch >= 1:
                for dd in scats[ch - 1]:
                    dd.wait()
            if ch + 1 < nch:
                loads[ch + 1] = issue_loads(ch + 1)
            for dd in loads[ch]:
                dd.wait()

            def grp(g, _, p=p):
                gd = g // 8
                off = (g % 8) * 16
                sv16 = sr[p][gd, pl.ds(off, 16)]
                dv = ds2[p][gd, pl.ds(off, 16)]
                kd = plsc.load_gather(an_v, [dv])
                sl = pl.ds(g * 16, 16)
                coef = (va[p][sl] + vb[p][sl]) / jnp.maximum(
                    ca[p][sl] + cb[p][sl], 1.0)
                qi2[p][gd, pl.ds(off, 16)] = kd * np2 + sv16
                cv2[p][gd, pl.ds(off, 16)] = coef
                return _

            lax.fori_loop(0, 128, grp, None)
            pend = []
            for k in range(16):
                pend.append(pltpu.async_copy(
                    cv2[p].at[k], q_sp.at[qi2[p].at[k]], ssems[p], add=True))
                pend.append(pltpu.async_copy(
                    ones_v, cnt_sp.at[sr[p].at[k]], ssems[p], add=True))
            scats[ch] = pend

        for dd in scats[nch - 1]:
            dd.wait()
        plsc.subcore_barrier()

        def qo(k, _):
            off = s * qt + k * 2560
            pltpu.sync_copy(q_sp.at[pl.ds(off, 2560)],
                            q_h.at[c].at[pl.ds(off, 2560)])
            return _

        lax.fori_loop(0, qt // 2560, qo, None)

        @pl.when(c == 0)
        def _cout():
            pltpu.sync_copy(cnt_sp.at[pl.ds(s * ct, ct)],
                            cnt_h.at[pl.ds(s * ct, ct)])

    return pl.kernel(
        body,
        out_type=(
            jax.ShapeDtypeStruct((_NC, qn), jnp.float32),
            jax.ShapeDtypeStruct((np2,), jnp.float32),
        ),
        mesh=mesh,
        compiler_params=pltpu.CompilerParams(needs_layout_passes=False),
        scratch_types=[
            pltpu.VMEM((n,), jnp.int32),
            pltpu.VMEM((16, 128), jnp.int32),
            pltpu.VMEM((16, 128), jnp.int32),
            pltpu.VMEM((16, 128), jnp.int32),
            pltpu.VMEM((16, 128), jnp.int32),
            pltpu.VMEM((2048,), jnp.float32),
            pltpu.VMEM((2048,), jnp.float32),
            pltpu.VMEM((2048,), jnp.float32),
            pltpu.VMEM((2048,), jnp.float32),
            pltpu.VMEM((2048,), jnp.float32),
            pltpu.VMEM((2048,), jnp.float32),
            pltpu.VMEM((2048,), jnp.float32),
            pltpu.VMEM((2048,), jnp.float32),
            pltpu.VMEM((16, 128), jnp.int32),
            pltpu.VMEM((16, 128), jnp.int32),
            pltpu.VMEM((16, 128), jnp.float32),
            pltpu.VMEM((16, 128), jnp.float32),
            pltpu.VMEM((128,), jnp.float32),
            pltpu.VMEM((2560,), jnp.float32),
            pltpu.SemaphoreType.DMA,
            pltpu.SemaphoreType.DMA,
            pltpu.SemaphoreType.DMA,
            pltpu.SemaphoreType.DMA,
            pltpu.VMEM_SHARED((qn,), jnp.float32),
            pltpu.VMEM_SHARED((np2,), jnp.float32),
        ],
    )(an, src2, dst2, sv, cp)


def _final_tc(q0f, q1f, v0, v1, cnt2, nap, np2):
    outf = v0.shape[1]

    def body(q0_ref, q1_ref, v0_ref, v1_ref, c_ref, o_ref):
        q0 = q0_ref[...].reshape(nap, np2)
        q1 = q1_ref[...].reshape(nap, np2)
        acc = lax.dot_general(q0, v0_ref[...], (((0,), (0,)), ((), ())),
                              preferred_element_type=jnp.float32)
        acc = acc + lax.dot_general(q1, v1_ref[...], (((0,), (0,)), ((), ())),
                                    preferred_element_type=jnp.float32)
        o_ref[...] = acc / jnp.maximum(c_ref[...], 1.0)

    return pl.pallas_call(
        body,
        out_shape=jax.ShapeDtypeStruct((np2, outf), jnp.float32),
    )(q0f, q1f, v0, v1, cnt2)


def kernel(atomic_number, edge_index, lg_edge_index, h, dnr,
           key_embedding, value_table, a, b, c, d):
    n = atomic_number.shape[0]
    e = edge_index.shape[1]
    t = lg_edge_index.shape[1]
    heads = a.shape[0]
    na = key_embedding.shape[0]
    hid = key_embedding.shape[1] // heads
    outf = value_table.shape[1] // heads

    ep = _round_up(e, _NS * 2048)
    tp = _round_up(t, _NC * _NS * 1024)
    np2 = _round_up(n + 1, _NS * 128)
    nap = _round_up(na, 8)

    an32 = atomic_number.astype(jnp.int32)
    src = edge_index[0].astype(jnp.int32)
    dst = edge_index[1].astype(jnp.int32)
    lsrc = lg_edge_index[0].astype(jnp.int32)
    ldst = lg_edge_index[1].astype(jnp.int32)
    src_p = jnp.pad(src, (0, ep - e))
    dst_p = jnp.pad(dst, (0, ep - e))
    src_p2 = jnp.pad(src, (0, ep - e), constant_values=n).reshape(
        ep // 128, 128)
    dst_p2 = dst_p.reshape(ep // 128, 128)
    lsrc2 = jnp.pad(lsrc, (0, tp - t), constant_values=e).reshape(
        tp // 128, 128)
    ldst2 = jnp.pad(ldst, (0, tp - t)).reshape(tp // 128, 128)

    h2 = h.reshape(t // 128, 128)
    dnr2 = dnr.reshape(t // 128, 128)
    kh = key_embedding.reshape(na, hid, heads).transpose(2, 0, 1)
    scal = jnp.stack([a, b % jnp.float32(np.pi), c, d])

    sp, w = _prep_tc(h2, dnr2, kh, scal)
    spf = sp.reshape(2, t)
    sp02 = jnp.pad(spf[0], (0, tp - t)).reshape(tp // 128, 128)
    sp12 = jnp.pad(spf[1], (0, tp - t)).reshape(tp // 128, 128)
    wflat = w.reshape(2 * na * na)

    sv, cp = _phase1_sc(an32, src_p, dst_p, lsrc2, ldst2, sp02, sp12, wflat,
                        n, na, e, ep, tp)
    q_out, cnt_out = _phase2_sc(an32, src_p2, dst_p2, sv, cp,
                                n, np2, nap, ep)

    v = value_table.reshape(na, outf, heads)
    v0p = jnp.pad(v[:, :, 0], ((0, nap - na), (0, 0)))
    v1p = jnp.pad(v[:, :, 1], ((0, nap - na), (0, 0)))
    cnt2 = cnt_out.reshape(np2, 1)
    out_full = _final_tc(q_out[0], q_out[1], v0p, v1p, cnt2, nap, np2)
    return out_full[:n]


# speedup vs baseline: 277.5611x; 1.2142x over previous
import numpy as np
import jax
import jax.numpy as jnp
from jax import lax
from jax.experimental import pallas as pl
from jax.experimental.pallas import tpu as pltpu
from jax.experimental.pallas import tpu_sc as plsc

_EPS = 1e-3
_NC, _NS = 2, 16


def _round_up(x, m):
    return (x + m - 1) // m * m


def _prep_tc(h2, dnr2, kh, scal):
    tch = h2.shape[0]
    na, hid = kh.shape[1], kh.shape[2]

    def body(h_ref, d_ref, k_ref, s_ref, sp_ref, w_ref):
        x = jnp.clip(h_ref[...], -_EPS, _EPS)
        theta = jnp.float32(np.pi / 2) - x - (x * x * x) * jnp.float32(1.0 / 6.0)
        d2 = d_ref[...] * d_ref[...]
        for hd in range(2):
            av = s_ref[0, hd]
            bv = s_ref[1, hd]
            cv = s_ref[2, hd]
            dv = s_ref[3, hd]
            ang = ((jnp.cos(av * theta + bv) + 1.0) * 0.5) ** cv
            rad = jnp.exp(-dv * d2)
            sp_ref[hd] = ang * rad
            k = k_ref[hd]
            w_ref[hd] = lax.dot_general(
                k, k, (((1,), (1,)), ((), ())),
                preferred_element_type=jnp.float32)

    return pl.pallas_call(
        body,
        in_specs=[
            pl.BlockSpec(memory_space=pltpu.VMEM),
            pl.BlockSpec(memory_space=pltpu.VMEM),
            pl.BlockSpec(memory_space=pltpu.VMEM),
            pl.BlockSpec(memory_space=pltpu.SMEM),
        ],
        out_shape=(
            jax.ShapeDtypeStruct((2, tch, 128), jnp.float32),
            jax.ShapeDtypeStruct((2, na, na), jnp.float32),
        ),
    )(h2, dnr2, kh, scal)


def _phase1_sc(an, src_p, dst_p, lsrc2, ldst2, sp02, sp12, wflat,
               nax, na, e, ep, tp):
    ept = ep // _NS
    tt = tp // (_NC * _NS)
    nch = tt // 1024
    mesh = plsc.VectorSubcoreMesh(core_axis_name="c", subcore_axis_name="s")

    def body(an_h, src_h, dst_h, lsrc_h, ldst_h, sp0_h, sp1_h, w_h,
             sv_h, cnt_h,
             an_v, w_v, srcbufA, srcbufB, idsbufA, idsbufB,
             lsA, lsB, ldA, ldB, s0A, s0B, s1A, s1B,
             kiA, kiB, kjA, kjB, v0A, v0B, v1A, v1B,
             ones_v, zbuf, lsemA, lsemB, gsemA, gsemB, ssemA, ssemB,
             ks_sh, kd_sh, s0_sh, s1_sh, c_sh):
        c = lax.axis_index("c")
        s = lax.axis_index("s")
        lsems = [lsemA, lsemB]
        ssems = [ssemA, ssemB]
        pltpu.sync_copy(an_h, an_v)
        pltpu.sync_copy(w_h, w_v)

        sbufs = [srcbufA, srcbufB]
        ibufs = [idsbufA, idsbufB]
        steps = ([(src_h, ks_sh, bi) for bi in range(ept // 2048)]
                 + [(dst_h, kd_sh, bi) for bi in range(ept // 2048)])
        nst = len(steps)

        def s1_load(i):
            eh, _, bi = steps[i]
            ebase = pl.multiple_of(s * ept + bi * 2048, 2048)
            return pltpu.async_copy(eh.at[pl.ds(ebase, 2048)],
                                    sbufs[i % 2], lsems[i % 2])

        ldd = {0: s1_load(0)}
        std = {}
        for i in range(nst):
            p = i % 2
            if i >= 2:
                std[i - 2].wait()
            if i + 1 < nst:
                ldd[i + 1] = s1_load(i + 1)
            ldd[i].wait()

            def gg(g, _, p=p):
                for u in range(4):
                    sl = pl.ds((g * 4 + u) * 16, 16)
                    ibufs[p][sl] = plsc.load_gather(an_v, [sbufs[p][sl]])
                return _

            lax.fori_loop(0, 32, gg, None)
            _, sh, bi = steps[i]
            ebase = pl.multiple_of(s * ept + bi * 2048, 2048)
            std[i] = pltpu.async_copy(ibufs[p], sh.at[pl.ds(ebase, 2048)],
                                      ssems[p])
        std[nst - 2].wait()
        std[nst - 1].wait()

        def zz(k, _):
            zbuf[pl.ds(k * 16, 16)] = jnp.zeros((16,), jnp.float32)
            return _

        lax.fori_loop(0, 128, zz, None)

        def oo(k, _):
            ones_v[pl.ds(k * 16, 16)] = jnp.ones((16,), jnp.float32)
            return _

        lax.fori_loop(0, 8, oo, None)

        def z2(k, _):
            off = s * (ep // _NS) + k * 2048
            pltpu.sync_copy(zbuf, s0_sh.at[pl.ds(off, 2048)])
            pltpu.sync_copy(zbuf, s1_sh.at[pl.ds(off, 2048)])
            pltpu.sync_copy(zbuf, c_sh.at[pl.ds(off, 2048)])
            return _

        lax.fori_loop(0, ep // _NS // 2048, z2, None)
        plsc.subcore_barrier()

        ls = [lsA, lsB]
        ld = [ldA, ldB]
        s0b = [s0A, s0B]
        s1b = [s1A, s1B]
        ki = [kiA, kiB]
        kj = [kjA, kjB]
        v0b = [v0A, v0B]
        v1b = [v1A, v1B]
        lsems = [lsemA, lsemB]
        gsems = [gsemA, gsemB]
        ssems = [ssemA, ssemB]
        rowbase = c * (tp // 2 // 128) + s * (tt // 128)

        def issue_loads(ch):
            p = ch % 2
            rb = pl.multiple_of(rowbase + ch * 8, 8)
            return [
                pltpu.async_copy(lsrc_h.at[pl.ds(rb, 8)], ls[p], lsems[p]),
                pltpu.async_copy(ldst_h.at[pl.ds(rb, 8)], ld[p], lsems[p]),
                pltpu.async_copy(sp0_h.at[pl.ds(rb, 8)], s0b[p], lsems[p]),
                pltpu.async_copy(sp1_h.at[pl.ds(rb, 8)], s1b[p], lsems[p]),
            ]

        def issue_gathers(p, si):
            gp = si % 2
            return [
                pltpu.async_copy(ks_sh.at[ls[p].at[si]], ki[gp], gsems[gp]),
                pltpu.async_copy(kd_sh.at[ld[p].at[si]], kj[gp], gsems[gp]),
            ]

        loads = {0: issue_loads(0)}
        scats = {}
        for ch in range(nch):
            p = ch % 2
            if ch >= 1:
                for dd in scats[ch - 1]:
                    dd.wait()
            if ch + 1 < nch:
                loads[ch + 1] = issue_loads(ch + 1)
            for dd in loads[ch]:
                dd.wait()
            gcur = issue_gathers(p, 0)
            pend = []
            for si in range(8):
                gp = si % 2
                if si < 7:
                    gnext = issue_gathers(p, si + 1)
                for dd in gcur:
                    dd.wait()

                def grp(g, _, si=si, gp=gp, p=p):
                    i = ki[gp][pl.ds(g * 16, 16)]
                    j = kj[gp][pl.ds(g * 16, 16)]
                    fidx = i * na + j
                    w0 = plsc.load_gather(w_v, [fidx])
                    w1 = plsc.load_gather(w_v, [fidx + na * na])
                    v0b[p][si, pl.ds(g * 16, 16)] = (
                        w0 * s0b[p][si, pl.ds(g * 16, 16)])
                    v1b[p][si, pl.ds(g * 16, 16)] = (
                        w1 * s1b[p][si, pl.ds(g * 16, 16)])
                    return _

                lax.fori_loop(0, 8, grp, None)
                pend.append(pltpu.async_copy(
                    v0b[p].at[si], s0_sh.at[ls[p].at[si]], ssems[p],
                    add=True))
                pend.append(pltpu.async_copy(
                    v1b[p].at[si], s1_sh.at[ls[p].at[si]], ssems[p],
                    add=True))
                pend.append(pltpu.async_copy(
                    ones_v, c_sh.at[ls[p].at[si]], ssems[p], add=True))
                gcur = gnext
            scats[ch] = pend
        for dd in scats[nch - 1]:
            dd.wait()
        plsc.subcore_barrier()

        ostart = s * (ep // _NS)

        def out(k, _):
            off = ostart + k * 1280
            pltpu.sync_copy(s0_sh.at[pl.ds(off, 1280)],
                            sv_h.at[c].at[0].at[pl.ds(off, 1280)])
            pltpu.sync_copy(s1_sh.at[pl.ds(off, 1280)],
                            sv_h.at[c].at[1].at[pl.ds(off, 1280)])
            pltpu.sync_copy(c_sh.at[pl.ds(off, 1280)],
                            cnt_h.at[c].at[pl.ds(off, 1280)])
            return _

        lax.fori_loop(0, ep // _NS // 1280, out, None)

    return pl.kernel(
        body,
        out_type=(
            jax.ShapeDtypeStruct((_NC, 2, ep), jnp.float32),
            jax.ShapeDtypeStruct((_NC, ep), jnp.float32),
        ),
        mesh=mesh,
        compiler_params=pltpu.CompilerParams(needs_layout_passes=False),
        scratch_types=[
            pltpu.VMEM((nax,), jnp.int32),
            pltpu.VMEM((2 * na * na,), jnp.float32),
            pltpu.VMEM((2048,), jnp.int32),
            pltpu.VMEM((2048,), jnp.int32),
            pltpu.VMEM((2048,), jnp.int32),
            pltpu.VMEM((2048,), jnp.int32),
            pltpu.VMEM((8, 128), jnp.int32),
            pltpu.VMEM((8, 128), jnp.int32),
            pltpu.VMEM((8, 128), jnp.int32),
            pltpu.VMEM((8, 128), jnp.int32),
            pltpu.VMEM((8, 128), jnp.float32),
            pltpu.VMEM((8, 128), jnp.float32),
            pltpu.VMEM((8, 128), jnp.float32),
            pltpu.VMEM((8, 128), jnp.float32),
            pltpu.VMEM((128,), jnp.int32),
            pltpu.VMEM((128,), jnp.int32),
            pltpu.VMEM((128,), jnp.int32),
            pltpu.VMEM((128,), jnp.int32),
            pltpu.VMEM((8, 128), jnp.float32),
            pltpu.VMEM((8, 128), jnp.float32),
            pltpu.VMEM((8, 128), jnp.float32),
            pltpu.VMEM((8, 128), jnp.float32),
            pltpu.VMEM((128,), jnp.float32),
            pltpu.VMEM((2048,), jnp.float32),
            pltpu.SemaphoreType.DMA,
            pltpu.SemaphoreType.DMA,
            pltpu.SemaphoreType.DMA,
            pltpu.SemaphoreType.DMA,
            pltpu.SemaphoreType.DMA,
            pltpu.SemaphoreType.DMA,
            pltpu.VMEM_SHARED((ep,), jnp.int32),
            pltpu.VMEM_SHARED((ep,), jnp.int32),
            pltpu.VMEM_SHARED((ep,), jnp.float32),
            pltpu.VMEM_SHARED((ep,), jnp.float32),
            pltpu.VMEM_SHARED((ep,), jnp.float32),
        ],
    )(an, src_p, dst_p, lsrc2, ldst2, sp02, sp12, wflat)


def _phase2_sc(an, src2, dst2, sv, cp, nax, np2, nap, ep):
    qn = nap * np2
    et = ep // _NS
    nch = et // 2048
    qt = qn // _NS
    ct = np2 // _NS
    mesh = plsc.VectorSubcoreMesh(core_axis_name="c", subcore_axis_name="s")

    def body(an_h, src_h, dst_h, sv_h, cp_h, q_h, cnt_h,
             an_v, srA, srB, dsA, dsB,
             vaA, vaB, vbA, vbB, caA, caB, cbA, cbB,
             qiA, qiB, cvA, cvB,
             ones_v, zq, lsemA, lsemB, ssemA, ssemB, q_sp, cnt_sp):
        c = lax.axis_index("c")
        s = lax.axis_index("s")
        pltpu.sync_copy(an_h, an_v)

        def zz(k, _):
            zq[pl.ds(k * 16, 16)] = jnp.zeros((16,), jnp.float32)
            return _

        lax.fori_loop(0, 160, zz, None)

        def oo(k, _):
            ones_v[pl.ds(k * 16, 16)] = jnp.ones((16,), jnp.float32)
            return _

        lax.fori_loop(0, 8, oo, None)

        def zql(k, _):
            pltpu.sync_copy(zq, q_sp.at[pl.ds(s * qt + k * 2560, 2560)])
            return _

        lax.fori_loop(0, qt // 2560, zql, None)
        pltpu.sync_copy(zq.at[pl.ds(0, ct)], cnt_sp.at[pl.ds(s * ct, ct)])
        plsc.subcore_barrier()

        sr = [srA, srB]
        ds2 = [dsA, dsB]
        va = [vaA, vaB]
        vb = [vbA, vbB]
        ca = [caA, caB]
        cb = [cbA, cbB]
        qi2 = [qiA, qiB]
        cv2 = [cvA, cvB]
        lsems = [lsemA, lsemB]
        ssems = [ssemA, ssemB]
        estart = s * et
        rstart = s * (et // 128)

        def issue_loads(ch):
            p = ch % 2
            eb = pl.multiple_of(estart + ch * 2048, 2048)
            rb = pl.multiple_of(rstart + ch * 16, 16)
            return [
                pltpu.async_copy(src_h.at[pl.ds(rb, 16)], sr[p], lsems[p]),
                pltpu.async_copy(dst_h.at[pl.ds(rb, 16)], ds2[p], lsems[p]),
                pltpu.async_copy(sv_h.at[0].at[c].at[pl.ds(eb, 2048)],
                                 va[p], lsems[p]),
                pltpu.async_copy(sv_h.at[1].at[c].at[pl.ds(eb, 2048)],
                                 vb[p], lsems[p]),
                pltpu.async_copy(cp_h.at[0].at[pl.ds(eb, 2048)],
                                 ca[p], lsems[p]),
                pltpu.async_copy(cp_h.at[1].at[pl.ds(eb, 2048)],
                                 cb[p], lsems[p]),
            ]

        loads = {0: issue_loads(0)}
        scats = {}
        for ch in range(nch):
            p = ch % 2
            if ch >= 1:
                for dd in scats[ch - 1]:
                    dd.wait()
            if ch + 1 < nch:
                loads[ch + 1] = issue_loads(ch + 1)
            for dd in loads[ch]:
                dd.wait()

            def grp(g, _, p=p):
                gd = g // 8
                off = (g % 8) * 16
                sv16 = sr[p][gd, pl.ds(off, 16)]
                dv = ds2[p][gd, pl.ds(off, 16)]
                kd = plsc.load_gather(an_v, [dv])
                sl = pl.ds(g * 16, 16)
                coef = (va[p][sl] + vb[p][sl]) / jnp.maximum(
                    ca[p][sl] + cb[p][sl], 1.0)
                qi2[p][gd, pl.ds(off, 16)] = kd * np2 + sv16
                cv2[p][gd, pl.ds(off, 16)] = coef
                return _

            lax.fori_loop(0, 128, grp, None)
            pend = []
            for k in range(16):
                pend.append(pltpu.async_copy(
                    cv2[p].at[k], q_sp.at[qi2[p].at[k]], ssems[p], add=True))
                pend.append(pltpu.async_copy(
                    ones_v, cnt_sp.at[sr[p].at[k]], ssems[p], add=True))
            scats[ch] = pend

        for dd in scats[nch - 1]:
            dd.wait()
        plsc.subcore_barrier()

        def qo(k, _):
            off = s * qt + k * 2560
            pltpu.sync_copy(q_sp.at[pl.ds(off, 2560)],
                            q_h.at[c].at[pl.ds(off, 2560)])
            return _

        lax.fori_loop(0, qt // 2560, qo, None)

        @pl.when(c == 0)
        def _cout():
            pltpu.sync_copy(cnt_sp.at[pl.ds(s * ct, ct)],
                            cnt_h.at[pl.ds(s * ct, ct)])

    return pl.kernel(
        body,
        out_type=(
            jax.ShapeDtypeStruct((_NC, qn), jnp.float32),
            jax.ShapeDtypeStruct((np2,), jnp.float32),
        ),
        mesh=mesh,
        compiler_params=pltpu.CompilerParams(needs_layout_passes=False),
        scratch_types=[
            pltpu.VMEM((nax,), jnp.int32),
            pltpu.VMEM((16, 128), jnp.int32),
            pltpu.VMEM((16, 128), jnp.int32),
            pltpu.VMEM((16, 128), jnp.int32),
            pltpu.VMEM((16, 128), jnp.int32),
            pltpu.VMEM((2048,), jnp.float32),
            pltpu.VMEM((2048,), jnp.float32),
            pltpu.VMEM((2048,), jnp.float32),
            pltpu.VMEM((2048,), jnp.float32),
            pltpu.VMEM((2048,), jnp.float32),
            pltpu.VMEM((2048,), jnp.float32),
            pltpu.VMEM((2048,), jnp.float32),
            pltpu.VMEM((2048,), jnp.float32),
            pltpu.VMEM((16, 128), jnp.int32),
            pltpu.VMEM((16, 128), jnp.int32),
            pltpu.VMEM((16, 128), jnp.float32),
            pltpu.VMEM((16, 128), jnp.float32),
            pltpu.VMEM((128,), jnp.float32),
            pltpu.VMEM((2560,), jnp.float32),
            pltpu.SemaphoreType.DMA,
            pltpu.SemaphoreType.DMA,
            pltpu.SemaphoreType.DMA,
            pltpu.SemaphoreType.DMA,
            pltpu.VMEM_SHARED((qn,), jnp.float32),
            pltpu.VMEM_SHARED((np2,), jnp.float32),
        ],
    )(an, src2, dst2, sv, cp)


def _final_tc(qf, v0, v1, cnt2, nap, np2):
    outf = v0.shape[1]

    def body(q_ref, v0_ref, v1_ref, c_ref, o_ref):
        q0 = q_ref[0].reshape(nap, np2)
        q1 = q_ref[1].reshape(nap, np2)
        acc = lax.dot_general(q0, v0_ref[...], (((0,), (0,)), ((), ())),
                              preferred_element_type=jnp.float32)
        acc = acc + lax.dot_general(q1, v1_ref[...], (((0,), (0,)), ((), ())),
                                    preferred_element_type=jnp.float32)
        o_ref[...] = acc / jnp.maximum(c_ref[...], 1.0)

    return pl.pallas_call(
        body,
        out_shape=jax.ShapeDtypeStruct((np2, outf), jnp.float32),
    )(qf, v0, v1, cnt2)


def kernel(atomic_number, edge_index, lg_edge_index, h, dnr,
           key_embedding, value_table, a, b, c, d):
    n = atomic_number.shape[0]
    e = edge_index.shape[1]
    t = lg_edge_index.shape[1]
    heads = a.shape[0]
    na = key_embedding.shape[0]
    hid = key_embedding.shape[1] // heads
    outf = value_table.shape[1] // heads

    ep = _round_up(e, _NS * 2048)
    tp = _round_up(t, _NC * _NS * 1024)
    np2 = _round_up(n + 1, _NS * 128)
    nap = _round_up(na, 8)

    an_p = jnp.pad(atomic_number.astype(jnp.int32), (0, np2 - n))
    src = edge_index[0].astype(jnp.int32)
    dst = edge_index[1].astype(jnp.int32)
    lsrc = lg_edge_index[0].astype(jnp.int32)
    ldst = lg_edge_index[1].astype(jnp.int32)
    src_p = jnp.pad(src, (0, ep - e), constant_values=n)
    dst_p = jnp.pad(dst, (0, ep - e))
    src_p2 = src_p.reshape(ep // 128, 128)
    dst_p2 = dst_p.reshape(ep // 128, 128)
    lsrc2 = jnp.pad(lsrc, (0, tp - t), constant_values=e).reshape(
        tp // 128, 128)
    ldst2 = jnp.pad(ldst, (0, tp - t)).reshape(tp // 128, 128)

    h2 = h.reshape(t // 128, 128)
    dnr2 = dnr.reshape(t // 128, 128)
    kh = key_embedding.reshape(na, hid, heads).transpose(2, 0, 1)
    scal = jnp.stack([a, b % jnp.float32(np.pi), c, d])

    sp, w = _prep_tc(h2, dnr2, kh, scal)
    spf = sp.reshape(2, t)
    sp02 = jnp.pad(spf[0], (0, tp - t)).reshape(tp // 128, 128)
    sp12 = jnp.pad(spf[1], (0, tp - t)).reshape(tp // 128, 128)
    wflat = w.reshape(2 * na * na)

    sv, cp = _phase1_sc(an_p, src_p, dst_p, lsrc2, ldst2, sp02, sp12, wflat,
                        np2, na, e, ep, tp)
    q_out, cnt_out = _phase2_sc(an_p, src_p2, dst_p2, sv, cp,
                                np2, np2, nap, ep)

    v = value_table.reshape(na, outf, heads)
    v0p = jnp.pad(v[:, :, 0], ((0, nap - na), (0, 0)))
    v1p = jnp.pad(v[:, :, 1], ((0, nap - na), (0, 0)))
    cnt2 = cnt_out.reshape(np2, 1)
    out_full = _final_tc(q_out, v0p, v1p, cnt2, nap, np2)
    return out_full[:n]


# speedup vs baseline: 316.5435x; 1.1404x over previous
import numpy as np
import jax
import jax.numpy as jnp
from jax import lax
from jax.experimental import pallas as pl
from jax.experimental.pallas import tpu as pltpu
from jax.experimental.pallas import tpu_sc as plsc

_EPS = 1e-3
_NC, _NS = 2, 16


def _round_up(x, m):
    return (x + m - 1) // m * m


def _prep_tc(h2, dnr2, kh, scal):
    tch = h2.shape[0]
    na, hid = kh.shape[1], kh.shape[2]

    def body(h_ref, d_ref, k_ref, s_ref, sp_ref, w_ref):
        x = jnp.clip(h_ref[...], -_EPS, _EPS)
        theta = jnp.float32(np.pi / 2) - x - (x * x * x) * jnp.float32(1.0 / 6.0)
        d2 = d_ref[...] * d_ref[...]
        for hd in range(2):
            av = s_ref[0, hd]
            bv = s_ref[1, hd]
            cv = s_ref[2, hd]
            dv = s_ref[3, hd]
            ang = ((jnp.cos(av * theta + bv) + 1.0) * 0.5) ** cv
            rad = jnp.exp(-dv * d2)
            sp_ref[hd] = ang * rad
            k = k_ref[hd]
            w_ref[hd] = lax.dot_general(
                k, k, (((1,), (1,)), ((), ())),
                preferred_element_type=jnp.float32)

    return pl.pallas_call(
        body,
        in_specs=[
            pl.BlockSpec(memory_space=pltpu.VMEM),
            pl.BlockSpec(memory_space=pltpu.VMEM),
            pl.BlockSpec(memory_space=pltpu.VMEM),
            pl.BlockSpec(memory_space=pltpu.SMEM),
        ],
        out_shape=(
            jax.ShapeDtypeStruct((2, tch, 128), jnp.float32),
            jax.ShapeDtypeStruct((2, na, na), jnp.float32),
        ),
    )(h2, dnr2, kh, scal)


def _phase1_sc(an, src_p, dst_p, lsrc2, ldst2, sp02, sp12, wflat,
               nax, na, e, ep, tp):
    ept = ep // _NS
    tt = tp // (_NC * _NS)
    nch = tt // 1024
    mesh = plsc.VectorSubcoreMesh(core_axis_name="c", subcore_axis_name="s")

    def body(an_h, src_h, dst_h, lsrc_h, ldst_h, sp0_h, sp1_h, w_h,
             sv_h, cnt_h,
             an_v, w_v, srcbufA, srcbufB, idsbufA, idsbufB,
             lsA, lsB, ldA, ldB, s0A, s0B, s1A, s1B,
             kiA, kiB, kjA, kjB, v0A, v0B, v1A, v1B,
             ones_v, zbuf, lsemA, lsemB, gsemA, gsemB, ssemA, ssemB, zsem,
             ks_sh, kd_sh, s0_sh, s1_sh, c_sh):
        c = lax.axis_index("c")
        s = lax.axis_index("s")
        lsems = [lsemA, lsemB]
        ssems = [ssemA, ssemB]
        pltpu.sync_copy(an_h, an_v)
        pltpu.sync_copy(w_h, w_v)

        def zz(k, _):
            zbuf[pl.ds(k * 16, 16)] = jnp.zeros((16,), jnp.float32)
            return _

        lax.fori_loop(0, 128, zz, None)

        def oo(k, _):
            ones_v[pl.ds(k * 16, 16)] = jnp.ones((16,), jnp.float32)
            return _

        lax.fori_loop(0, 8, oo, None)
        zdescs = []
        for k in range(ep // _NS // 2048):
            off = pl.multiple_of(s * (ep // _NS) + k * 2048, 2048)
            for sh in (s0_sh, s1_sh, c_sh):
                zdescs.append(
                    pltpu.async_copy(zbuf, sh.at[pl.ds(off, 2048)], zsem))

        sbufs = [srcbufA, srcbufB]
        ibufs = [idsbufA, idsbufB]
        steps = ([(src_h, ks_sh, bi) for bi in range(ept // 2048)]
                 + [(dst_h, kd_sh, bi) for bi in range(ept // 2048)])
        nst = len(steps)

        def s1_load(i):
            eh, _, bi = steps[i]
            ebase = pl.multiple_of(s * ept + bi * 2048, 2048)
            return pltpu.async_copy(eh.at[pl.ds(ebase, 2048)],
                                    sbufs[i % 2], lsems[i % 2])

        ldd = {0: s1_load(0)}
        std = {}
        for i in range(nst):
            p = i % 2
            if i >= 2:
                std[i - 2].wait()
            if i + 1 < nst:
                ldd[i + 1] = s1_load(i + 1)
            ldd[i].wait()

            def gg(g, _, p=p):
                for u in range(4):
                    sl = pl.ds((g * 4 + u) * 16, 16)
                    ibufs[p][sl] = plsc.load_gather(an_v, [sbufs[p][sl]])
                return _

            lax.fori_loop(0, 32, gg, None)
            _, sh, bi = steps[i]
            ebase = pl.multiple_of(s * ept + bi * 2048, 2048)
            std[i] = pltpu.async_copy(ibufs[p], sh.at[pl.ds(ebase, 2048)],
                                      ssems[p])
        std[nst - 2].wait()
        std[nst - 1].wait()
        for dd in zdescs:
            dd.wait()
        plsc.subcore_barrier()

        ls = [lsA, lsB]
        ld = [ldA, ldB]
        s0b = [s0A, s0B]
        s1b = [s1A, s1B]
        ki = [kiA, kiB]
        kj = [kjA, kjB]
        v0b = [v0A, v0B]
        v1b = [v1A, v1B]
        lsems = [lsemA, lsemB]
        gsems = [gsemA, gsemB]
        ssems = [ssemA, ssemB]
        rowbase = c * (tp // 2 // 128) + s * (tt // 128)

        def issue_loads(ch):
            p = ch % 2
            rb = pl.multiple_of(rowbase + ch * 8, 8)
            return [
                pltpu.async_copy(lsrc_h.at[pl.ds(rb, 8)], ls[p], lsems[p]),
                pltpu.async_copy(ldst_h.at[pl.ds(rb, 8)], ld[p], lsems[p]),
                pltpu.async_copy(sp0_h.at[pl.ds(rb, 8)], s0b[p], lsems[p]),
                pltpu.async_copy(sp1_h.at[pl.ds(rb, 8)], s1b[p], lsems[p]),
            ]

        def issue_gathers(p, si):
            gp = si % 2
            return [
                pltpu.async_copy(ks_sh.at[ls[p].at[si]], ki[gp], gsems[gp]),
                pltpu.async_copy(kd_sh.at[ld[p].at[si]], kj[gp], gsems[gp]),
            ]

        loads = {0: issue_loads(0)}
        scats = {}
        for ch in range(nch):
            p = ch % 2
            if ch >= 1:
                for dd in scats[ch - 1]:
                    dd.wait()
            if ch + 1 < nch:
                loads[ch + 1] = issue_loads(ch + 1)
            for dd in loads[ch]:
                dd.wait()
            gcur = issue_gathers(p, 0)
            pend = []
            for si in range(8):
                gp = si % 2
                if si < 7:
                    gnext = issue_gathers(p, si + 1)
                for dd in gcur:
                    dd.wait()

                def grp(g, _, si=si, gp=gp, p=p):
                    i = ki[gp][pl.ds(g * 16, 16)]
                    j = kj[gp][pl.ds(g * 16, 16)]
                    fidx = i * na + j
                    w0 = plsc.load_gather(w_v, [fidx])
                    w1 = plsc.load_gather(w_v, [fidx + na * na])
                    v0b[p][si, pl.ds(g * 16, 16)] = (
                        w0 * s0b[p][si, pl.ds(g * 16, 16)])
                    v1b[p][si, pl.ds(g * 16, 16)] = (
                        w1 * s1b[p][si, pl.ds(g * 16, 16)])
                    return _

                lax.fori_loop(0, 8, grp, None)
                pend.append(pltpu.async_copy(
                    v0b[p].at[si], s0_sh.at[ls[p].at[si]], ssems[p],
                    add=True))
                pend.append(pltpu.async_copy(
                    v1b[p].at[si], s1_sh.at[ls[p].at[si]], ssems[p],
                    add=True))
                pend.append(pltpu.async_copy(
                    ones_v, c_sh.at[ls[p].at[si]], ssems[p], add=True))
                gcur = gnext
            scats[ch] = pend
        for dd in scats[nch - 1]:
            dd.wait()
        plsc.subcore_barrier()

        odescs = []
        for k in range(ep // _NS // 1280):
            off = pl.multiple_of(s * (ep // _NS) + k * 1280, 1280)
            sl = pl.ds(off, 1280)
            odescs.append(pltpu.async_copy(
                s0_sh.at[sl], sv_h.at[c].at[0].at[sl], ssemA))
            odescs.append(pltpu.async_copy(
                s1_sh.at[sl], sv_h.at[c].at[1].at[sl], ssemB))
            odescs.append(pltpu.async_copy(
                c_sh.at[sl], cnt_h.at[c].at[sl], zsem))
        for dd in odescs:
            dd.wait()

    return pl.kernel(
        body,
        out_type=(
            jax.ShapeDtypeStruct((_NC, 2, ep), jnp.float32),
            jax.ShapeDtypeStruct((_NC, ep), jnp.float32),
        ),
        mesh=mesh,
        compiler_params=pltpu.CompilerParams(needs_layout_passes=False),
        scratch_types=[
            pltpu.VMEM((nax,), jnp.int32),
            pltpu.VMEM((2 * na * na,), jnp.float32),
            pltpu.VMEM((2048,), jnp.int32),
            pltpu.VMEM((2048,), jnp.int32),
            pltpu.VMEM((2048,), jnp.int32),
            pltpu.VMEM((2048,), jnp.int32),
            pltpu.VMEM((8, 128), jnp.int32),
            pltpu.VMEM((8, 128), jnp.int32),
            pltpu.VMEM((8, 128), jnp.int32),
            pltpu.VMEM((8, 128), jnp.int32),
            pltpu.VMEM((8, 128), jnp.float32),
            pltpu.VMEM((8, 128), jnp.float32),
            pltpu.VMEM((8, 128), jnp.float32),
            pltpu.VMEM((8, 128), jnp.float32),
            pltpu.VMEM((128,), jnp.int32),
            pltpu.VMEM((128,), jnp.int32),
            pltpu.VMEM((128,), jnp.int32),
            pltpu.VMEM((128,), jnp.int32),
            pltpu.VMEM((8, 128), jnp.float32),
            pltpu.VMEM((8, 128), jnp.float32),
            pltpu.VMEM((8, 128), jnp.float32),
            pltpu.VMEM((8, 128), jnp.float32),
            pltpu.VMEM((128,), jnp.float32),
            pltpu.VMEM((2048,), jnp.float32),
            pltpu.SemaphoreType.DMA,
            pltpu.SemaphoreType.DMA,
            pltpu.SemaphoreType.DMA,
            pltpu.SemaphoreType.DMA,
            pltpu.SemaphoreType.DMA,
            pltpu.SemaphoreType.DMA,
            pltpu.SemaphoreType.DMA,
            pltpu.VMEM_SHARED((ep,), jnp.int32),
            pltpu.VMEM_SHARED((ep,), jnp.int32),
            pltpu.VMEM_SHARED((ep,), jnp.float32),
            pltpu.VMEM_SHARED((ep,), jnp.float32),
            pltpu.VMEM_SHARED((ep,), jnp.float32),
        ],
    )(an, src_p, dst_p, lsrc2, ldst2, sp02, sp12, wflat)


def _phase2_sc(an, src2, dst2, sv, cp, nax, np2, nap, ep):
    qn = nap * np2
    et = ep // _NS
    nch = et // 2048
    qt = qn // _NS
    ct = np2 // _NS
    mesh = plsc.VectorSubcoreMesh(core_axis_name="c", subcore_axis_name="s")

    def body(an_h, src_h, dst_h, sv_h, cp_h, q_h, cnt_h,
             an_v, srA, srB, dsA, dsB,
             vaA, vaB, vbA, vbB, caA, caB, cbA, cbB,
             qiA, qiB, cvA, cvB,
             ones_v, zq, lsemA, lsemB, ssemA, ssemB, zsem, q_sp, cnt_sp):
        c = lax.axis_index("c")
        s = lax.axis_index("s")
        pltpu.sync_copy(an_h, an_v)

        def zz(k, _):
            zq[pl.ds(k * 16, 16)] = jnp.zeros((16,), jnp.float32)
            return _

        lax.fori_loop(0, 160, zz, None)

        def oo(k, _):
            ones_v[pl.ds(k * 16, 16)] = jnp.ones((16,), jnp.float32)
            return _

        lax.fori_loop(0, 8, oo, None)
        zdescs = [pltpu.async_copy(zq.at[pl.ds(0, ct)],
                                   cnt_sp.at[pl.ds(s * ct, ct)], zsem)]
        for k in range(qt // 2560):
            off = pl.multiple_of(s * qt + k * 2560, 2560)
            zdescs.append(pltpu.async_copy(zq, q_sp.at[pl.ds(off, 2560)],
                                           zsem))
        for dd in zdescs:
            dd.wait()
        plsc.subcore_barrier()

        sr = [srA, srB]
        ds2 = [dsA, dsB]
        va = [vaA, vaB]
        vb = [vbA, vbB]
        ca = [caA, caB]
        cb = [cbA, cbB]
        qi2 = [qiA, qiB]
        cv2 = [cvA, cvB]
        lsems = [lsemA, lsemB]
        ssems = [ssemA, ssemB]
        estart = s * et
        rstart = s * (et // 128)

        def issue_loads(ch):
            p = ch % 2
            eb = pl.multiple_of(estart + ch * 2048, 2048)
            rb = pl.multiple_of(rstart + ch * 16, 16)
            return [
                pltpu.async_copy(src_h.at[pl.ds(rb, 16)], sr[p], lsems[p]),
                pltpu.async_copy(dst_h.at[pl.ds(rb, 16)], ds2[p], lsems[p]),
                pltpu.async_copy(sv_h.at[0].at[c].at[pl.ds(eb, 2048)],
                                 va[p], lsems[p]),
                pltpu.async_copy(sv_h.at[1].at[c].at[pl.ds(eb, 2048)],
                                 vb[p], lsems[p]),
                pltpu.async_copy(cp_h.at[0].at[pl.ds(eb, 2048)],
                                 ca[p], lsems[p]),
                pltpu.async_copy(cp_h.at[1].at[pl.ds(eb, 2048)],
                                 cb[p], lsems[p]),
            ]

        loads = {0: issue_loads(0)}
        scats = {}
        for ch in range(nch):
            p = ch % 2
            if ch >= 1:
                for dd in scats[ch - 1]:
                    dd.wait()
            if ch + 1 < nch:
                loads[ch + 1] = issue_loads(ch + 1)
            for dd in loads[ch]:
                dd.wait()

            def grp(g, _, p=p):
                gd = g // 8
                off = (g % 8) * 16
                sv16 = sr[p][gd, pl.ds(off, 16)]
                dv = ds2[p][gd, pl.ds(off, 16)]
                kd = plsc.load_gather(an_v, [dv])
                sl = pl.ds(g * 16, 16)
                coef = (va[p][sl] + vb[p][sl]) / jnp.maximum(
                    ca[p][sl] + cb[p][sl], 1.0)
                qi2[p][gd, pl.ds(off, 16)] = kd * np2 + sv16
                cv2[p][gd, pl.ds(off, 16)] = coef
                return _

            lax.fori_loop(0, 128, grp, None)
            pend = []
            for k in range(16):
                pend.append(pltpu.async_copy(
                    cv2[p].at[k], q_sp.at[qi2[p].at[k]], ssems[p], add=True))
                pend.append(pltpu.async_copy(
                    ones_v, cnt_sp.at[sr[p].at[k]], ssems[p], add=True))
            scats[ch] = pend

        for dd in scats[nch - 1]:
            dd.wait()
        plsc.subcore_barrier()

        odescs = []
        for k in range(qt // 2560):
            off = pl.multiple_of(s * qt + k * 2560, 2560)
            odescs.append(pltpu.async_copy(
                q_sp.at[pl.ds(off, 2560)], q_h.at[c].at[pl.ds(off, 2560)],
                ssemA if k % 2 == 0 else ssemB))
        for dd in odescs:
            dd.wait()

        @pl.when(c == 0)
        def _cout():
            pltpu.sync_copy(cnt_sp.at[pl.ds(s * ct, ct)],
                            cnt_h.at[pl.ds(s * ct, ct)])

    return pl.kernel(
        body,
        out_type=(
            jax.ShapeDtypeStruct((_NC, qn), jnp.float32),
            jax.ShapeDtypeStruct((np2,), jnp.float32),
        ),
        mesh=mesh,
        compiler_params=pltpu.CompilerParams(needs_layout_passes=False),
        scratch_types=[
            pltpu.VMEM((nax,), jnp.int32),
            pltpu.VMEM((16, 128), jnp.int32),
            pltpu.VMEM((16, 128), jnp.int32),
            pltpu.VMEM((16, 128), jnp.int32),
            pltpu.VMEM((16, 128), jnp.int32),
            pltpu.VMEM((2048,), jnp.float32),
            pltpu.VMEM((2048,), jnp.float32),
            pltpu.VMEM((2048,), jnp.float32),
            pltpu.VMEM((2048,), jnp.float32),
            pltpu.VMEM((2048,), jnp.float32),
            pltpu.VMEM((2048,), jnp.float32),
            pltpu.VMEM((2048,), jnp.float32),
            pltpu.VMEM((2048,), jnp.float32),
            pltpu.VMEM((16, 128), jnp.int32),
            pltpu.VMEM((16, 128), jnp.int32),
            pltpu.VMEM((16, 128), jnp.float32),
            pltpu.VMEM((16, 128), jnp.float32),
            pltpu.VMEM((128,), jnp.float32),
            pltpu.VMEM((2560,), jnp.float32),
            pltpu.SemaphoreType.DMA,
            pltpu.SemaphoreType.DMA,
            pltpu.SemaphoreType.DMA,
            pltpu.SemaphoreType.DMA,
            pltpu.SemaphoreType.DMA,
            pltpu.VMEM_SHARED((qn,), jnp.float32),
            pltpu.VMEM_SHARED((np2,), jnp.float32),
        ],
    )(an, src2, dst2, sv, cp)


def _final_tc(qf, v0, v1, cnt2, nap, np2):
    outf = v0.shape[1]

    def body(q_ref, v0_ref, v1_ref, c_ref, o_ref):
        q0 = q_ref[0].reshape(nap, np2)
        q1 = q_ref[1].reshape(nap, np2)
        acc = lax.dot_general(q0, v0_ref[...], (((0,), (0,)), ((), ())),
                              preferred_element_type=jnp.float32)
        acc = acc + lax.dot_general(q1, v1_ref[...], (((0,), (0,)), ((), ())),
                                    preferred_element_type=jnp.float32)
        o_ref[...] = acc / jnp.maximum(c_ref[...], 1.0)

    return pl.pallas_call(
        body,
        out_shape=jax.ShapeDtypeStruct((np2, outf), jnp.float32),
    )(qf, v0, v1, cnt2)


def kernel(atomic_number, edge_index, lg_edge_index, h, dnr,
           key_embedding, value_table, a, b, c, d):
    n = atomic_number.shape[0]
    e = edge_index.shape[1]
    t = lg_edge_index.shape[1]
    heads = a.shape[0]
    na = key_embedding.shape[0]
    hid = key_embedding.shape[1] // heads
    outf = value_table.shape[1] // heads

    ep = _round_up(e, _NS * 2048)
    tp = _round_up(t, _NC * _NS * 1024)
    np2 = _round_up(n + 1, _NS * 128)
    nap = _round_up(na, 8)

    an_p = jnp.pad(atomic_number.astype(jnp.int32), (0, np2 - n))
    src = edge_index[0].astype(jnp.int32)
    dst = edge_index[1].astype(jnp.int32)
    lsrc = lg_edge_index[0].astype(jnp.int32)
    ldst = lg_edge_index[1].astype(jnp.int32)
    src_p = jnp.pad(src, (0, ep - e), constant_values=n)
    dst_p = jnp.pad(dst, (0, ep - e))
    src_p2 = src_p.reshape(ep // 128, 128)
    dst_p2 = dst_p.reshape(ep // 128, 128)
    lsrc2 = jnp.pad(lsrc, (0, tp - t), constant_values=e).reshape(
        tp // 128, 128)
    ldst2 = jnp.pad(ldst, (0, tp - t)).reshape(tp // 128, 128)

    h2 = h.reshape(t // 128, 128)
    dnr2 = dnr.reshape(t // 128, 128)
    kh = key_embedding.reshape(na, hid, heads).transpose(2, 0, 1)
    scal = jnp.stack([a, b % jnp.float32(np.pi), c, d])

    sp, w = _prep_tc(h2, dnr2, kh, scal)
    spf = sp.reshape(2, t)
    sp02 = jnp.pad(spf[0], (0, tp - t)).reshape(tp // 128, 128)
    sp12 = jnp.pad(spf[1], (0, tp - t)).reshape(tp // 128, 128)
    wflat = w.reshape(2 * na * na)

    sv, cp = _phase1_sc(an_p, src_p, dst_p, lsrc2, ldst2, sp02, sp12, wflat,
                        np2, na, e, ep, tp)
    q_out, cnt_out = _phase2_sc(an_p, src_p2, dst_p2, sv, cp,
                                np2, np2, nap, ep)

    v = value_table.reshape(na, outf, heads)
    v0p = jnp.pad(v[:, :, 0], ((0, nap - na), (0, 0)))
    v1p = jnp.pad(v[:, :, 1], ((0, nap - na), (0, 0)))
    cnt2 = cnt_out.reshape(np2, 1)
    out_full = _final_tc(q_out, v0p, v1p, cnt2, nap, np2)
    return out_full[:n]
